# baseline (speedup 1.0000x reference)
import functools

import jax
import jax.numpy as jnp
from jax import lax
from jax.experimental import pallas as pl
from jax.experimental.pallas import tpu as pltpu
from jax.experimental.pallas import tpu_sc as plsc

N_NODES = 50000
N_PAD = 50176
T_STEPS = 12
F_IN = 4
HOR = 3
E_EDGES = 800000
E_PAD = 819200
IDX_ROWS = E_PAD // 128
NC = 2
NS = 16
ROWS_PER_TILE = N_PAD // NS
KB = 8
KB_S2 = 4

BN = 1024
NBLK = N_PAD // BN

_mesh = plsc.VectorSubcoreMesh(
    core_axis_name="c", subcore_axis_name="s", num_cores=NC, num_subcores=NS)


def _al8(v):
    return pl.multiple_of(v, 8)


def _zero_fill_1d(buf, n):
    z = jnp.zeros((16,), jnp.float32)

    def step(i, _):
        buf[pl.ds(i * 16, 16)] = z
        return 0

    lax.fori_loop(0, n // 16, step, 0)


def _zero_fill_2d(buf, rows, cols):
    z = jnp.zeros((16,), jnp.float32)

    def step(i, _):
        for c0 in range(0, cols, 16):
            buf[i, pl.ds(c0, 16)] = z
        return 0

    lax.fori_loop(0, rows, step, 0)


def _deg_body(dst_hbm, out_hbm, deg_sh, zbuf, ones_v, idx_v):
    cid = lax.axis_index("c")
    sid = lax.axis_index("s")
    _zero_fill_1d(zbuf, ROWS_PER_TILE)
    one = jnp.ones((16,), jnp.float32)
    for i in range(8):
        ones_v[pl.ds(i * 16, 16)] = one
    lo = _al8(sid * ROWS_PER_TILE)
    pltpu.sync_copy(zbuf, deg_sh.at[pl.ds(lo, ROWS_PER_TILE)])
    plsc.subcore_barrier()

    w = cid * NS + sid
    n_rows = IDX_ROWS // (NC * NS)

    def blk(b, _):
        r0 = _al8(w * n_rows + b * KB)
        pltpu.sync_copy(dst_hbm.at[pl.ds(r0, KB)], idx_v)
        for j in range(KB):
            pltpu.sync_copy(ones_v, deg_sh.at[idx_v.at[j]], add=True)
        return 0

    lax.fori_loop(0, n_rows // KB, blk, 0)
    plsc.subcore_barrier()
    pltpu.sync_copy(deg_sh.at[pl.ds(lo, ROWS_PER_TILE)], zbuf)
    pltpu.sync_copy(zbuf,
                    out_hbm.at[pl.ds(_al8(cid * N_PAD + lo), ROWS_PER_TILE)])


_deg_call = functools.partial(
    pl.kernel,
    _deg_body,
    out_type=jax.ShapeDtypeStruct((NC * N_PAD,), jnp.float32),
    mesh=_mesh,
    scratch_types=[
        pltpu.VMEM_SHARED((N_PAD,), jnp.float32),
        pltpu.VMEM((ROWS_PER_TILE,), jnp.float32),
        pltpu.VMEM((128,), jnp.float32),
        pltpu.VMEM((KB, 128), jnp.int32),
    ],
    compiler_params=pltpu.CompilerParams(use_tc_tiling_on_sc=False),
)()


def _s1_body(tab_hbm, src_hbm, dst_hbm, out_hbm,
             acc_sh, zbuf, srcv, dstv, rows_v, sem):
    cid = lax.axis_index("c")
    sid = lax.axis_index("s")
    _zero_fill_2d(zbuf, ROWS_PER_TILE // 4, 16)
    lo = _al8(sid * ROWS_PER_TILE)
    for q in range(4):
        pltpu.sync_copy(zbuf, acc_sh.at[pl.ds(
            _al8(lo + q * (ROWS_PER_TILE // 4)), ROWS_PER_TILE // 4)])
    plsc.subcore_barrier()

    w = cid * NS + sid
    n_rows = IDX_ROWS // (NC * NS)

    def blk(b, _):
        r0 = _al8(w * n_rows + b * KB)
        pltpu.sync_copy(src_hbm.at[pl.ds(_al8(r0 * 128), KB * 128)], srcv)
        pltpu.sync_copy(dst_hbm.at[pl.ds(r0, KB)], dstv)
        pltpu.async_copy(tab_hbm.at[srcv], rows_v, sem).wait()
        for j in range(KB):
            pltpu.sync_copy(rows_v.at[pl.ds(j * 128, 128)],
                            acc_sh.at[dstv.at[j]], add=True)
        return 0

    lax.fori_loop(0, n_rows // KB, blk, 0)
    plsc.subcore_barrier()
    for q in range(4):
        qlo = _al8(lo + q * (ROWS_PER_TILE // 4))
        pltpu.sync_copy(acc_sh.at[pl.ds(qlo, ROWS_PER_TILE // 4)], zbuf)
        pltpu.sync_copy(zbuf, out_hbm.at[cid, pl.ds(qlo, ROWS_PER_TILE // 4)])


_s1_call = functools.partial(
    pl.kernel,
    _s1_body,
    out_type=jax.ShapeDtypeStruct((NC, N_PAD, 16), jnp.float32),
    mesh=_mesh,
    scratch_types=[
        pltpu.VMEM_SHARED((N_PAD, 16), jnp.float32),
        pltpu.VMEM((ROWS_PER_TILE // 4, 16), jnp.float32),
        pltpu.VMEM((KB * 128,), jnp.int32),
        pltpu.VMEM((KB, 128), jnp.int32),
        pltpu.VMEM((KB * 128, 16), jnp.float32),
        pltpu.SemaphoreType.DMA,
    ],
    compiler_params=pltpu.CompilerParams(use_tc_tiling_on_sc=False),
)()


def _s2_body(tab_hbm, src2_hbm, dst_hbm, out_hbm,
             acc_sh, zbuf, srcv, dstv, rows_v, sem):
    cid = lax.axis_index("c")
    sid = lax.axis_index("s")
    _zero_fill_2d(zbuf, 112, 32)
    lo = _al8(sid * ROWS_PER_TILE)
    for q in range(28):
        pltpu.sync_copy(zbuf, acc_sh.at[pl.ds(_al8(lo + q * 112), 112)])
    plsc.subcore_barrier()

    n_rows = IDX_ROWS // NS

    def blk(b, _):
        r0 = pl.multiple_of(sid * n_rows + b * KB_S2, 4)
        pltpu.sync_copy(src2_hbm.at[cid, pl.ds(_al8(r0 * 128), KB_S2 * 128)],
                        srcv)
        pltpu.sync_copy(dst_hbm.at[pl.ds(r0, KB_S2)], dstv)
        pltpu.async_copy(tab_hbm.at[srcv], rows_v, sem).wait()
        for j in range(KB_S2):
            pltpu.sync_copy(rows_v.at[pl.ds(j * 128, 128)],
                            acc_sh.at[dstv.at[j]], add=True)
        return 0

    lax.fori_loop(0, n_rows // KB_S2, blk, 0)
    plsc.subcore_barrier()
    for q in range(28):
        qlo = _al8(lo + q * 112)
        pltpu.sync_copy(acc_sh.at[pl.ds(qlo, 112)], zbuf)
        pltpu.sync_copy(zbuf, out_hbm.at[cid, pl.ds(qlo, 112)])


_s2_call = functools.partial(
    pl.kernel,
    _s2_body,
    out_type=jax.ShapeDtypeStruct((NC, N_PAD, 32), jnp.float32),
    mesh=_mesh,
    scratch_types=[
        pltpu.VMEM_SHARED((N_PAD, 32), jnp.float32),
        pltpu.VMEM((112, 32), jnp.float32),
        pltpu.VMEM((KB_S2 * 128,), jnp.int32),
        pltpu.VMEM((KB_S2, 128), jnp.int32),
        pltpu.VMEM((KB_S2 * 128, 32), jnp.float32),
        pltpu.SemaphoreType.DMA,
    ],
    compiler_params=pltpu.CompilerParams(use_tc_tiling_on_sc=False),
)()


def _prep_body(deg_ref, x12_ref, xd_ref, dinv_ref):
    deg = deg_ref[0, :] + deg_ref[1, :] + 1.0
    dinv = lax.rsqrt(deg)
    dinv_ref[...] = dinv
    xd12 = x12_ref[...] * dinv[:, None]
    xd_ref[...] = jnp.concatenate(
        [xd12, jnp.zeros((BN, 4), jnp.float32)], axis=1)


def _prep_call(deg2, x12):
    return pl.pallas_call(
        _prep_body,
        grid=(NBLK,),
        in_specs=[
            pl.BlockSpec((NC, BN), lambda i: (0, i)),
            pl.BlockSpec((BN, 12), lambda i: (i, 0)),
        ],
        out_specs=[
            pl.BlockSpec((BN, 16), lambda i: (i, 0)),
            pl.BlockSpec((BN,), lambda i: (i,)),
        ],
        out_shape=[
            jax.ShapeDtypeStruct((N_PAD, 16), jnp.float32),
            jax.ShapeDtypeStruct((N_PAD,), jnp.float32),
        ],
    )(deg2, x12)


def _mid_body(s1_ref, xd_ref, dinv_ref, w1_ref, b1_ref, k_ref, tb1_ref,
              x2d_ref):
    dinv = dinv_ref[...][:, None]
    y = (s1_ref[0] + s1_ref[1] + xd_ref[...]) * dinv
    w1 = w1_ref[...]
    b1 = b1_ref[...]

    def gcn(t):
        return jnp.maximum(
            jnp.dot(y[:, 4 * t:4 * t + 4], w1,
                    preferred_element_type=jnp.float32) + b1, 0.0)

    g9, g10, g11 = gcn(0), gcn(1), gcn(2)
    k0, k1, k2 = k_ref[0], k_ref[1], k_ref[2]
    tb1 = tb1_ref[...]
    o10 = jnp.maximum(
        jnp.dot(g9, k0, preferred_element_type=jnp.float32)
        + jnp.dot(g10, k1, preferred_element_type=jnp.float32)
        + jnp.dot(g11, k2, preferred_element_type=jnp.float32) + tb1, 0.0)
    o11 = jnp.maximum(
        jnp.dot(g10, k0, preferred_element_type=jnp.float32)
        + jnp.dot(g11, k1, preferred_element_type=jnp.float32) + tb1, 0.0)
    x2d_ref[0] = o10 * dinv
    x2d_ref[1] = o11 * dinv


def _mid_call(s1, xd16, dinv, w1, b1, tw1t, tb1):
    return pl.pallas_call(
        _mid_body,
        grid=(NBLK,),
        in_specs=[
            pl.BlockSpec((NC, BN, 16), lambda i: (0, i, 0)),
            pl.BlockSpec((BN, 16), lambda i: (i, 0)),
            pl.BlockSpec((BN,), lambda i: (i,)),
            pl.BlockSpec((F_IN, 32), lambda i: (0, 0)),
            pl.BlockSpec((32,), lambda i: (0,)),
            pl.BlockSpec((3, 32, 32), lambda i: (0, 0, 0)),
            pl.BlockSpec((32,), lambda i: (0,)),
        ],
        out_specs=pl.BlockSpec((2, BN, 32), lambda i: (0, i, 0)),
        out_shape=jax.ShapeDtypeStruct((2, N_PAD, 32), jnp.float32),
    )(s1, xd16, dinv, w1, b1, tw1t, tb1)


def _head_body(s2_ref, x2d_ref, dinv_ref, w2_ref, b2_ref, q_ref, tb2_ref,
               fw1_ref, fb1_ref, fw2_ref, fb2_ref, out_ref):
    dinv = dinv_ref[...][:, None]
    y10 = (s2_ref[0] + x2d_ref[0]) * dinv
    y11 = (s2_ref[1] + x2d_ref[1]) * dinv
    w2 = w2_ref[...]
    b2 = b2_ref[...]
    g10 = jnp.maximum(
        jnp.dot(y10, w2, preferred_element_type=jnp.float32) + b2, 0.0)
    g11 = jnp.maximum(
        jnp.dot(y11, w2, preferred_element_type=jnp.float32) + b2, 0.0)
    h = jnp.maximum(
        jnp.dot(g10, q_ref[0], preferred_element_type=jnp.float32)
        + jnp.dot(g11, q_ref[1], preferred_element_type=jnp.float32)
        + tb2_ref[...], 0.0)
    f = jnp.maximum(
        jnp.dot(h, fw1_ref[...], preferred_element_type=jnp.float32)
        + fb1_ref[...], 0.0)
    p = jnp.dot(f, fw2_ref[...], preferred_element_type=jnp.float32) \
        + fb2_ref[...]
    out_ref[...] = jnp.maximum(p, 0.0) + jnp.log1p(jnp.exp(-jnp.abs(p)))


def _head_call(s2, x2d, dinv, w2, b2, tw2t, tb2, fw1, fb1, fw2, fb2):
    return pl.pallas_call(
        _head_body,
        grid=(NBLK,),
        in_specs=[
            pl.BlockSpec((NC, BN, 32), lambda i: (0, i, 0)),
            pl.BlockSpec((2, BN, 32), lambda i: (0, i, 0)),
            pl.BlockSpec((BN,), lambda i: (i,)),
            pl.BlockSpec((32, 64), lambda i: (0, 0)),
            pl.BlockSpec((64,), lambda i: (0,)),
            pl.BlockSpec((2, 64, 64), lambda i: (0, 0, 0)),
            pl.BlockSpec((64,), lambda i: (0,)),
            pl.BlockSpec((64, 64), lambda i: (0, 0)),
            pl.BlockSpec((64,), lambda i: (0,)),
            pl.BlockSpec((64, HOR * F_IN), lambda i: (0, 0)),
            pl.BlockSpec((HOR * F_IN,), lambda i: (0,)),
        ],
        out_specs=pl.BlockSpec((BN, HOR * F_IN), lambda i: (i, 0)),
        out_shape=jax.ShapeDtypeStruct((N_PAD, HOR * F_IN), jnp.float32),
    )(s2, x2d, dinv, w2, b2, tw2t, tb2, fw1, fb1, fw2, fb2)


def kernel(x, edge_index, W1, b1, tw1, tb1, W2, b2, tw2, tb2,
           fw1, fb1, fw2, fb2):
    n = x.shape[0]
    x12 = x[:, T_STEPS - 3:, :].reshape(n, 3 * F_IN)
    x12 = jnp.pad(x12, ((0, N_PAD - n), (0, 0)))

    pad_n = E_PAD - E_EDGES
    ar = jnp.arange(pad_n, dtype=jnp.int32)
    src_flat = jnp.concatenate([edge_index[0], ar % N_NODES])
    dstp = jnp.concatenate(
        [edge_index[1], N_NODES + (ar % (N_PAD - N_NODES))]).reshape(
        IDX_ROWS, 128)
    src2 = jnp.stack([src_flat, src_flat + N_PAD])

    tw1t = jnp.transpose(tw1, (2, 1, 0))
    tw2t = jnp.transpose(tw2, (2, 1, 0))[:2]

    deg2 = _deg_call(dstp).reshape(NC, N_PAD)
    xd16, dinv = _prep_call(deg2, x12)
    s1 = _s1_call(xd16, src_flat, dstp)
    x2d = _mid_call(s1, xd16, dinv, W1, b1, tw1t, tb1)
    tab2 = x2d.reshape(2 * N_PAD, 32)
    s2 = _s2_call(tab2, src2, dstp)
    out = _head_call(s2, x2d, dinv, W2, b2, tw2t, tb2, fw1, fb1, fw2, fb2)
    return out[:n].reshape(n, HOR, F_IN)


# speedup vs baseline: 1.0827x; 1.0827x over previous
import functools

import jax
import jax.numpy as jnp
from jax import lax
from jax.experimental import pallas as pl
from jax.experimental.pallas import tpu as pltpu
from jax.experimental.pallas import tpu_sc as plsc

N_NODES = 50000
N_PAD = 50176
T_STEPS = 12
F_IN = 4
HOR = 3
E_EDGES = 800000
E_PAD = 819200
IDX_ROWS = E_PAD // 128
NC = 2
NS = 16
ROWS_PER_TILE = N_PAD // NS
KB = 8
KB_S2 = 4

BN = 7168
NBLK = N_PAD // BN

_mesh = plsc.VectorSubcoreMesh(
    core_axis_name="c", subcore_axis_name="s", num_cores=NC, num_subcores=NS)


def _al8(v):
    return pl.multiple_of(v, 8)


def _zero_fill_1d(buf, n):
    z = jnp.zeros((16,), jnp.float32)

    def step(i, _):
        buf[pl.ds(i * 16, 16)] = z
        return 0

    lax.fori_loop(0, n // 16, step, 0)


def _zero_fill_2d(buf, rows, cols):
    z = jnp.zeros((16,), jnp.float32)

    def step(i, _):
        for c0 in range(0, cols, 16):
            buf[i, pl.ds(c0, 16)] = z
        return 0

    lax.fori_loop(0, rows, step, 0)


def _deg_body(dst_hbm, out_hbm, deg_sh, zbuf, ones_v, idx_v):
    cid = lax.axis_index("c")
    sid = lax.axis_index("s")
    _zero_fill_1d(zbuf, ROWS_PER_TILE)
    one = jnp.ones((16,), jnp.float32)
    for i in range(8):
        ones_v[pl.ds(i * 16, 16)] = one
    lo = _al8(sid * ROWS_PER_TILE)
    pltpu.sync_copy(zbuf, deg_sh.at[pl.ds(lo, ROWS_PER_TILE)])
    plsc.subcore_barrier()

    w = cid * NS + sid
    n_rows = IDX_ROWS // (NC * NS)

    def blk(b, _):
        r0 = _al8(w * n_rows + b * KB)
        pltpu.sync_copy(dst_hbm.at[pl.ds(r0, KB)], idx_v)
        for j in range(KB):
            pltpu.sync_copy(ones_v, deg_sh.at[idx_v.at[j]], add=True)
        return 0

    lax.fori_loop(0, n_rows // KB, blk, 0)
    plsc.subcore_barrier()
    pltpu.sync_copy(deg_sh.at[pl.ds(lo, ROWS_PER_TILE)], zbuf)
    pltpu.sync_copy(zbuf,
                    out_hbm.at[pl.ds(_al8(cid * N_PAD + lo), ROWS_PER_TILE)])


_deg_call = functools.partial(
    pl.kernel,
    _deg_body,
    out_type=jax.ShapeDtypeStruct((NC * N_PAD,), jnp.float32),
    mesh=_mesh,
    scratch_types=[
        pltpu.VMEM_SHARED((N_PAD,), jnp.float32),
        pltpu.VMEM((ROWS_PER_TILE,), jnp.float32),
        pltpu.VMEM((128,), jnp.float32),
        pltpu.VMEM((KB, 128), jnp.int32),
    ],
    compiler_params=pltpu.CompilerParams(use_tc_tiling_on_sc=False),
)()


def _s1_body(tab_hbm, src_hbm, dst_hbm, out_hbm,
             acc_sh, zbuf, srcv, dstv, rows_v, sem):
    cid = lax.axis_index("c")
    sid = lax.axis_index("s")
    _zero_fill_2d(zbuf, ROWS_PER_TILE // 4, 16)
    lo = _al8(sid * ROWS_PER_TILE)
    for q in range(4):
        pltpu.sync_copy(zbuf, acc_sh.at[pl.ds(
            _al8(lo + q * (ROWS_PER_TILE // 4)), ROWS_PER_TILE // 4)])
    plsc.subcore_barrier()

    w = cid * NS + sid
    n_rows = IDX_ROWS // (NC * NS)

    def blk(b, _):
        r0 = _al8(w * n_rows + b * KB)
        pltpu.sync_copy(src_hbm.at[pl.ds(_al8(r0 * 128), KB * 128)], srcv)
        pltpu.sync_copy(dst_hbm.at[pl.ds(r0, KB)], dstv)
        pltpu.async_copy(tab_hbm.at[srcv], rows_v, sem).wait()
        for j in range(KB):
            pltpu.sync_copy(rows_v.at[pl.ds(j * 128, 128)],
                            acc_sh.at[dstv.at[j]], add=True)
        return 0

    lax.fori_loop(0, n_rows // KB, blk, 0)
    plsc.subcore_barrier()
    for q in range(4):
        qlo = _al8(lo + q * (ROWS_PER_TILE // 4))
        pltpu.sync_copy(acc_sh.at[pl.ds(qlo, ROWS_PER_TILE // 4)], zbuf)
        pltpu.sync_copy(zbuf, out_hbm.at[cid, pl.ds(qlo, ROWS_PER_TILE // 4)])


_s1_call = functools.partial(
    pl.kernel,
    _s1_body,
    out_type=jax.ShapeDtypeStruct((NC, N_PAD, 16), jnp.float32),
    mesh=_mesh,
    scratch_types=[
        pltpu.VMEM_SHARED((N_PAD, 16), jnp.float32),
        pltpu.VMEM((ROWS_PER_TILE // 4, 16), jnp.float32),
        pltpu.VMEM((KB * 128,), jnp.int32),
        pltpu.VMEM((KB, 128), jnp.int32),
        pltpu.VMEM((KB * 128, 16), jnp.float32),
        pltpu.SemaphoreType.DMA,
    ],
    compiler_params=pltpu.CompilerParams(use_tc_tiling_on_sc=False),
)()


def _s2_body(tab_hbm, src2_hbm, dst_hbm, out_hbm,
             acc_sh, zbuf, srcv, dstv, rows_v, sem):
    cid = lax.axis_index("c")
    sid = lax.axis_index("s")
    _zero_fill_2d(zbuf, 112, 32)
    lo = _al8(sid * ROWS_PER_TILE)
    for q in range(28):
        pltpu.sync_copy(zbuf, acc_sh.at[pl.ds(_al8(lo + q * 112), 112)])
    plsc.subcore_barrier()

    n_rows = IDX_ROWS // NS

    def blk(b, _):
        r0 = pl.multiple_of(sid * n_rows + b * KB_S2, 4)
        pltpu.sync_copy(src2_hbm.at[cid, pl.ds(_al8(r0 * 128), KB_S2 * 128)],
                        srcv)
        pltpu.sync_copy(dst_hbm.at[pl.ds(r0, KB_S2)], dstv)
        pltpu.async_copy(tab_hbm.at[srcv], rows_v, sem).wait()
        for j in range(KB_S2):
            pltpu.sync_copy(rows_v.at[pl.ds(j * 128, 128)],
                            acc_sh.at[dstv.at[j]], add=True)
        return 0

    lax.fori_loop(0, n_rows // KB_S2, blk, 0)
    plsc.subcore_barrier()
    for q in range(28):
        qlo = _al8(lo + q * 112)
        pltpu.sync_copy(acc_sh.at[pl.ds(qlo, 112)], zbuf)
        pltpu.sync_copy(zbuf, out_hbm.at[cid, pl.ds(qlo, 112)])


_s2_call = functools.partial(
    pl.kernel,
    _s2_body,
    out_type=jax.ShapeDtypeStruct((NC, N_PAD, 32), jnp.float32),
    mesh=_mesh,
    scratch_types=[
        pltpu.VMEM_SHARED((N_PAD, 32), jnp.float32),
        pltpu.VMEM((112, 32), jnp.float32),
        pltpu.VMEM((KB_S2 * 128,), jnp.int32),
        pltpu.VMEM((KB_S2, 128), jnp.int32),
        pltpu.VMEM((KB_S2 * 128, 32), jnp.float32),
        pltpu.SemaphoreType.DMA,
    ],
    compiler_params=pltpu.CompilerParams(use_tc_tiling_on_sc=False),
)()


def _prep_body(deg_ref, x12_ref, xd_ref, dinv_ref):
    deg = deg_ref[0, :] + deg_ref[1, :] + 1.0
    dinv = lax.rsqrt(deg)
    dinv_ref[...] = dinv
    xd12 = x12_ref[...] * dinv[:, None]
    xd_ref[...] = jnp.concatenate(
        [xd12, jnp.zeros((BN, 4), jnp.float32)], axis=1)


def _prep_call(deg2, x12):
    return pl.pallas_call(
        _prep_body,
        grid=(NBLK,),
        in_specs=[
            pl.BlockSpec((NC, BN), lambda i: (0, i)),
            pl.BlockSpec((BN, 12), lambda i: (i, 0)),
        ],
        out_specs=[
            pl.BlockSpec((BN, 16), lambda i: (i, 0)),
            pl.BlockSpec((BN,), lambda i: (i,)),
        ],
        out_shape=[
            jax.ShapeDtypeStruct((N_PAD, 16), jnp.float32),
            jax.ShapeDtypeStruct((N_PAD,), jnp.float32),
        ],
    )(deg2, x12)


def _mid_body(s1_ref, xd_ref, dinv_ref, w1_ref, b1_ref, k_ref, tb1_ref,
              x2d_ref):
    dinv = dinv_ref[...][:, None]
    y = (s1_ref[0] + s1_ref[1] + xd_ref[...]) * dinv
    w1 = w1_ref[...]
    b1 = b1_ref[...]

    def gcn(t):
        return jnp.maximum(
            jnp.dot(y[:, 4 * t:4 * t + 4], w1,
                    preferred_element_type=jnp.float32) + b1, 0.0)

    g9, g10, g11 = gcn(0), gcn(1), gcn(2)
    k0, k1, k2 = k_ref[0], k_ref[1], k_ref[2]
    tb1 = tb1_ref[...]
    o10 = jnp.maximum(
        jnp.dot(g9, k0, preferred_element_type=jnp.float32)
        + jnp.dot(g10, k1, preferred_element_type=jnp.float32)
        + jnp.dot(g11, k2, preferred_element_type=jnp.float32) + tb1, 0.0)
    o11 = jnp.maximum(
        jnp.dot(g10, k0, preferred_element_type=jnp.float32)
        + jnp.dot(g11, k1, preferred_element_type=jnp.float32) + tb1, 0.0)
    x2d_ref[0] = o10 * dinv
    x2d_ref[1] = o11 * dinv


def _mid_call(s1, xd16, dinv, w1, b1, tw1t, tb1):
    return pl.pallas_call(
        _mid_body,
        grid=(NBLK,),
        in_specs=[
            pl.BlockSpec((NC, BN, 16), lambda i: (0, i, 0)),
            pl.BlockSpec((BN, 16), lambda i: (i, 0)),
            pl.BlockSpec((BN,), lambda i: (i,)),
            pl.BlockSpec((F_IN, 32), lambda i: (0, 0)),
            pl.BlockSpec((32,), lambda i: (0,)),
            pl.BlockSpec((3, 32, 32), lambda i: (0, 0, 0)),
            pl.BlockSpec((32,), lambda i: (0,)),
        ],
        out_specs=pl.BlockSpec((2, BN, 32), lambda i: (0, i, 0)),
        out_shape=jax.ShapeDtypeStruct((2, N_PAD, 32), jnp.float32),
    )(s1, xd16, dinv, w1, b1, tw1t, tb1)


def _head_body(s2_ref, x2d_ref, dinv_ref, w2_ref, b2_ref, q_ref, tb2_ref,
               fw1_ref, fb1_ref, fw2_ref, fb2_ref, out_ref):
    dinv = dinv_ref[...][:, None]
    y10 = (s2_ref[0] + x2d_ref[0]) * dinv
    y11 = (s2_ref[1] + x2d_ref[1]) * dinv
    w2 = w2_ref[...]
    b2 = b2_ref[...]
    g10 = jnp.maximum(
        jnp.dot(y10, w2, preferred_element_type=jnp.float32) + b2, 0.0)
    g11 = jnp.maximum(
        jnp.dot(y11, w2, preferred_element_type=jnp.float32) + b2, 0.0)
    h = jnp.maximum(
        jnp.dot(g10, q_ref[0], preferred_element_type=jnp.float32)
        + jnp.dot(g11, q_ref[1], preferred_element_type=jnp.float32)
        + tb2_ref[...], 0.0)
    f = jnp.maximum(
        jnp.dot(h, fw1_ref[...], preferred_element_type=jnp.float32)
        + fb1_ref[...], 0.0)
    p = jnp.dot(f, fw2_ref[...], preferred_element_type=jnp.float32) \
        + fb2_ref[...]
    out_ref[...] = jnp.maximum(p, 0.0) + jnp.log1p(jnp.exp(-jnp.abs(p)))


def _head_call(s2, x2d, dinv, w2, b2, tw2t, tb2, fw1, fb1, fw2, fb2):
    return pl.pallas_call(
        _head_body,
        grid=(NBLK,),
        in_specs=[
            pl.BlockSpec((NC, BN, 32), lambda i: (0, i, 0)),
            pl.BlockSpec((2, BN, 32), lambda i: (0, i, 0)),
            pl.BlockSpec((BN,), lambda i: (i,)),
            pl.BlockSpec((32, 64), lambda i: (0, 0)),
            pl.BlockSpec((64,), lambda i: (0,)),
            pl.BlockSpec((2, 64, 64), lambda i: (0, 0, 0)),
            pl.BlockSpec((64,), lambda i: (0,)),
            pl.BlockSpec((64, 64), lambda i: (0, 0)),
            pl.BlockSpec((64,), lambda i: (0,)),
            pl.BlockSpec((64, HOR * F_IN), lambda i: (0, 0)),
            pl.BlockSpec((HOR * F_IN,), lambda i: (0,)),
        ],
        out_specs=pl.BlockSpec((BN, HOR * F_IN), lambda i: (i, 0)),
        out_shape=jax.ShapeDtypeStruct((N_PAD, HOR * F_IN), jnp.float32),
    )(s2, x2d, dinv, w2, b2, tw2t, tb2, fw1, fb1, fw2, fb2)


def kernel(x, edge_index, W1, b1, tw1, tb1, W2, b2, tw2, tb2,
           fw1, fb1, fw2, fb2):
    n = x.shape[0]
    x12 = x[:, T_STEPS - 3:, :].reshape(n, 3 * F_IN)
    x12 = jnp.pad(x12, ((0, N_PAD - n), (0, 0)))

    pad_n = E_PAD - E_EDGES
    ar = jnp.arange(pad_n, dtype=jnp.int32)
    src_flat = jnp.concatenate([edge_index[0], ar % N_NODES])
    dstp = jnp.concatenate(
        [edge_index[1], N_NODES + (ar % (N_PAD - N_NODES))]).reshape(
        IDX_ROWS, 128)
    src2 = jnp.stack([src_flat, src_flat + N_PAD])

    tw1t = jnp.transpose(tw1, (2, 1, 0))
    tw2t = jnp.transpose(tw2, (2, 1, 0))[:2]

    deg2 = _deg_call(dstp).reshape(NC, N_PAD)
    xd16, dinv = _prep_call(deg2, x12)
    s1 = _s1_call(xd16, src_flat, dstp)
    x2d = _mid_call(s1, xd16, dinv, W1, b1, tw1t, tb1)
    tab2 = x2d.reshape(2 * N_PAD, 32)
    s2 = _s2_call(tab2, src2, dstp)
    out = _head_call(s2, x2d, dinv, W2, b2, tw2t, tb2, fw1, fb1, fw2, fb2)
    return out[:n].reshape(n, HOR, F_IN)


# speedup vs baseline: 1.1369x; 1.0500x over previous
import functools

import jax
import jax.numpy as jnp
from jax import lax
from jax.experimental import pallas as pl
from jax.experimental.pallas import tpu as pltpu
from jax.experimental.pallas import tpu_sc as plsc

N_NODES = 50000
N_PAD = 50176
T_STEPS = 12
F_IN = 4
HOR = 3
E_EDGES = 800000
E_PAD = 819200
IDX_ROWS = E_PAD // 128
NC = 2
NS = 16
ROWS_PER_TILE = N_PAD // NS
KB = 8
KB_S2 = 4

BN = 7168
NBLK = N_PAD // BN

_mesh = plsc.VectorSubcoreMesh(
    core_axis_name="c", subcore_axis_name="s", num_cores=NC, num_subcores=NS)


def _al8(v):
    return pl.multiple_of(v, 8)


def _zero_fill_1d(buf, n):
    z = jnp.zeros((16,), jnp.float32)

    def step(i, _):
        buf[pl.ds(i * 16, 16)] = z
        return 0

    lax.fori_loop(0, n // 16, step, 0)


def _zero_fill_2d(buf, rows, cols):
    z = jnp.zeros((16,), jnp.float32)

    def step(i, _):
        for c0 in range(0, cols, 16):
            buf[i, pl.ds(c0, 16)] = z
        return 0

    lax.fori_loop(0, rows, step, 0)


def _deg_body(dst_hbm, out_hbm, deg_sh, zbuf, ones_v, idx_v, ssem):
    cid = lax.axis_index("c")
    sid = lax.axis_index("s")
    _zero_fill_1d(zbuf, ROWS_PER_TILE)
    one = jnp.ones((16,), jnp.float32)
    for i in range(8):
        ones_v[pl.ds(i * 16, 16)] = one
    lo = _al8(sid * ROWS_PER_TILE)
    pltpu.sync_copy(zbuf, deg_sh.at[pl.ds(lo, ROWS_PER_TILE)])
    plsc.subcore_barrier()

    w = cid * NS + sid
    n_rows = IDX_ROWS // (NC * NS)

    def blk(b, _):
        r0 = _al8(w * n_rows + b * KB)
        pltpu.sync_copy(dst_hbm.at[pl.ds(r0, KB)], idx_v)
        for j in range(KB):
            pltpu.async_copy(ones_v, deg_sh.at[idx_v.at[j]], ssem, add=True)
        for j in range(KB):
            pltpu.make_async_copy(ones_v, deg_sh.at[idx_v.at[j]],
                                  ssem).wait()
        return 0

    lax.fori_loop(0, n_rows // KB, blk, 0)
    plsc.subcore_barrier()
    pltpu.sync_copy(deg_sh.at[pl.ds(lo, ROWS_PER_TILE)], zbuf)
    pltpu.sync_copy(zbuf,
                    out_hbm.at[pl.ds(_al8(cid * N_PAD + lo), ROWS_PER_TILE)])


_deg_call = functools.partial(
    pl.kernel,
    _deg_body,
    out_type=jax.ShapeDtypeStruct((NC * N_PAD,), jnp.float32),
    mesh=_mesh,
    scratch_types=[
        pltpu.VMEM_SHARED((N_PAD,), jnp.float32),
        pltpu.VMEM((ROWS_PER_TILE,), jnp.float32),
        pltpu.VMEM((128,), jnp.float32),
        pltpu.VMEM((KB, 128), jnp.int32),
        pltpu.SemaphoreType.DMA,
    ],
    compiler_params=pltpu.CompilerParams(use_tc_tiling_on_sc=False),
)()


def _s1_body(tab_hbm, src_hbm, dst_hbm, out_hbm,
             acc_sh, zbuf, srcv, dstv, rows_v, sem, ssem):
    cid = lax.axis_index("c")
    sid = lax.axis_index("s")
    _zero_fill_2d(zbuf, ROWS_PER_TILE // 4, 16)
    lo = _al8(sid * ROWS_PER_TILE)
    for q in range(4):
        pltpu.sync_copy(zbuf, acc_sh.at[pl.ds(
            _al8(lo + q * (ROWS_PER_TILE // 4)), ROWS_PER_TILE // 4)])
    plsc.subcore_barrier()

    w = cid * NS + sid
    n_rows = IDX_ROWS // (NC * NS)

    def blk(b, _):
        r0 = _al8(w * n_rows + b * KB)
        pltpu.sync_copy(src_hbm.at[pl.ds(_al8(r0 * 128), KB * 128)], srcv)
        pltpu.sync_copy(dst_hbm.at[pl.ds(r0, KB)], dstv)
        pltpu.async_copy(tab_hbm.at[srcv], rows_v, sem).wait()
        for j in range(KB):
            pltpu.async_copy(rows_v.at[pl.ds(j * 128, 128)],
                             acc_sh.at[dstv.at[j]], ssem, add=True)
        for j in range(KB):
            pltpu.make_async_copy(rows_v.at[pl.ds(j * 128, 128)],
                                  acc_sh.at[dstv.at[j]], ssem).wait()
        return 0

    lax.fori_loop(0, n_rows // KB, blk, 0)
    plsc.subcore_barrier()
    for q in range(4):
        qlo = _al8(lo + q * (ROWS_PER_TILE // 4))
        pltpu.sync_copy(acc_sh.at[pl.ds(qlo, ROWS_PER_TILE // 4)], zbuf)
        pltpu.sync_copy(zbuf, out_hbm.at[cid, pl.ds(qlo, ROWS_PER_TILE // 4)])


_s1_call = functools.partial(
    pl.kernel,
    _s1_body,
    out_type=jax.ShapeDtypeStruct((NC, N_PAD, 16), jnp.float32),
    mesh=_mesh,
    scratch_types=[
        pltpu.VMEM_SHARED((N_PAD, 16), jnp.float32),
        pltpu.VMEM((ROWS_PER_TILE // 4, 16), jnp.float32),
        pltpu.VMEM((KB * 128,), jnp.int32),
        pltpu.VMEM((KB, 128), jnp.int32),
        pltpu.VMEM((KB * 128, 16), jnp.float32),
        pltpu.SemaphoreType.DMA,
        pltpu.SemaphoreType.DMA,
    ],
    compiler_params=pltpu.CompilerParams(use_tc_tiling_on_sc=False),
)()


def _s2_body(tab_hbm, src2_hbm, dst_hbm, out_hbm,
             acc_sh, zbuf, srcv, dstv, rows_v, sem, ssem):
    cid = lax.axis_index("c")
    sid = lax.axis_index("s")
    _zero_fill_2d(zbuf, 112, 32)
    lo = _al8(sid * ROWS_PER_TILE)
    for q in range(28):
        pltpu.sync_copy(zbuf, acc_sh.at[pl.ds(_al8(lo + q * 112), 112)])
    plsc.subcore_barrier()

    n_rows = IDX_ROWS // NS

    def blk(b, _):
        r0 = pl.multiple_of(sid * n_rows + b * KB_S2, 4)
        pltpu.sync_copy(src2_hbm.at[cid, pl.ds(_al8(r0 * 128), KB_S2 * 128)],
                        srcv)
        pltpu.sync_copy(dst_hbm.at[pl.ds(r0, KB_S2)], dstv)
        pltpu.async_copy(tab_hbm.at[srcv], rows_v, sem).wait()
        for j in range(KB_S2):
            pltpu.async_copy(rows_v.at[pl.ds(j * 128, 128)],
                             acc_sh.at[dstv.at[j]], ssem, add=True)
        for j in range(KB_S2):
            pltpu.make_async_copy(rows_v.at[pl.ds(j * 128, 128)],
                                  acc_sh.at[dstv.at[j]], ssem).wait()
        return 0

    lax.fori_loop(0, n_rows // KB_S2, blk, 0)
    plsc.subcore_barrier()
    for q in range(28):
        qlo = _al8(lo + q * 112)
        pltpu.sync_copy(acc_sh.at[pl.ds(qlo, 112)], zbuf)
        pltpu.sync_copy(zbuf, out_hbm.at[cid, pl.ds(qlo, 112)])


_s2_call = functools.partial(
    pl.kernel,
    _s2_body,
    out_type=jax.ShapeDtypeStruct((NC, N_PAD, 32), jnp.float32),
    mesh=_mesh,
    scratch_types=[
        pltpu.VMEM_SHARED((N_PAD, 32), jnp.float32),
        pltpu.VMEM((112, 32), jnp.float32),
        pltpu.VMEM((KB_S2 * 128,), jnp.int32),
        pltpu.VMEM((KB_S2, 128), jnp.int32),
        pltpu.VMEM((KB_S2 * 128, 32), jnp.float32),
        pltpu.SemaphoreType.DMA,
        pltpu.SemaphoreType.DMA,
    ],
    compiler_params=pltpu.CompilerParams(use_tc_tiling_on_sc=False),
)()


def _prep_body(deg_ref, x12_ref, xd_ref, dinv_ref):
    deg = deg_ref[0, :] + deg_ref[1, :] + 1.0
    dinv = lax.rsqrt(deg)
    dinv_ref[...] = dinv
    xd12 = x12_ref[...] * dinv[:, None]
    xd_ref[...] = jnp.concatenate(
        [xd12, jnp.zeros((BN, 4), jnp.float32)], axis=1)


def _prep_call(deg2, x12):
    return pl.pallas_call(
        _prep_body,
        grid=(NBLK,),
        in_specs=[
            pl.BlockSpec((NC, BN), lambda i: (0, i)),
            pl.BlockSpec((BN, 12), lambda i: (i, 0)),
        ],
        out_specs=[
            pl.BlockSpec((BN, 16), lambda i: (i, 0)),
            pl.BlockSpec((BN,), lambda i: (i,)),
        ],
        out_shape=[
            jax.ShapeDtypeStruct((N_PAD, 16), jnp.float32),
            jax.ShapeDtypeStruct((N_PAD,), jnp.float32),
        ],
    )(deg2, x12)


def _mid_body(s1_ref, xd_ref, dinv_ref, w1_ref, b1_ref, k_ref, tb1_ref,
              x2d_ref):
    dinv = dinv_ref[...][:, None]
    y = (s1_ref[0] + s1_ref[1] + xd_ref[...]) * dinv
    w1 = w1_ref[...]
    b1 = b1_ref[...]

    def gcn(t):
        return jnp.maximum(
            jnp.dot(y[:, 4 * t:4 * t + 4], w1,
                    preferred_element_type=jnp.float32) + b1, 0.0)

    g9, g10, g11 = gcn(0), gcn(1), gcn(2)
    k0, k1, k2 = k_ref[0], k_ref[1], k_ref[2]
    tb1 = tb1_ref[...]
    o10 = jnp.maximum(
        jnp.dot(g9, k0, preferred_element_type=jnp.float32)
        + jnp.dot(g10, k1, preferred_element_type=jnp.float32)
        + jnp.dot(g11, k2, preferred_element_type=jnp.float32) + tb1, 0.0)
    o11 = jnp.maximum(
        jnp.dot(g10, k0, preferred_element_type=jnp.float32)
        + jnp.dot(g11, k1, preferred_element_type=jnp.float32) + tb1, 0.0)
    x2d_ref[0] = o10 * dinv
    x2d_ref[1] = o11 * dinv


def _mid_call(s1, xd16, dinv, w1, b1, tw1t, tb1):
    return pl.pallas_call(
        _mid_body,
        grid=(NBLK,),
        in_specs=[
            pl.BlockSpec((NC, BN, 16), lambda i: (0, i, 0)),
            pl.BlockSpec((BN, 16), lambda i: (i, 0)),
            pl.BlockSpec((BN,), lambda i: (i,)),
            pl.BlockSpec((F_IN, 32), lambda i: (0, 0)),
            pl.BlockSpec((32,), lambda i: (0,)),
            pl.BlockSpec((3, 32, 32), lambda i: (0, 0, 0)),
            pl.BlockSpec((32,), lambda i: (0,)),
        ],
        out_specs=pl.BlockSpec((2, BN, 32), lambda i: (0, i, 0)),
        out_shape=jax.ShapeDtypeStruct((2, N_PAD, 32), jnp.float32),
    )(s1, xd16, dinv, w1, b1, tw1t, tb1)


def _head_body(s2_ref, x2d_ref, dinv_ref, w2_ref, b2_ref, q_ref, tb2_ref,
               fw1_ref, fb1_ref, fw2_ref, fb2_ref, out_ref):
    dinv = dinv_ref[...][:, None]
    y10 = (s2_ref[0] + x2d_ref[0]) * dinv
    y11 = (s2_ref[1] + x2d_ref[1]) * dinv
    w2 = w2_ref[...]
    b2 = b2_ref[...]
    g10 = jnp.maximum(
        jnp.dot(y10, w2, preferred_element_type=jnp.float32) + b2, 0.0)
    g11 = jnp.maximum(
        jnp.dot(y11, w2, preferred_element_type=jnp.float32) + b2, 0.0)
    h = jnp.maximum(
        jnp.dot(g10, q_ref[0], preferred_element_type=jnp.float32)
        + jnp.dot(g11, q_ref[1], preferred_element_type=jnp.float32)
        + tb2_ref[...], 0.0)
    f = jnp.maximum(
        jnp.dot(h, fw1_ref[...], preferred_element_type=jnp.float32)
        + fb1_ref[...], 0.0)
    p = jnp.dot(f, fw2_ref[...], preferred_element_type=jnp.float32) \
        + fb2_ref[...]
    out_ref[...] = jnp.maximum(p, 0.0) + jnp.log1p(jnp.exp(-jnp.abs(p)))


def _head_call(s2, x2d, dinv, w2, b2, tw2t, tb2, fw1, fb1, fw2, fb2):
    return pl.pallas_call(
        _head_body,
        grid=(NBLK,),
        in_specs=[
            pl.BlockSpec((NC, BN, 32), lambda i: (0, i, 0)),
            pl.BlockSpec((2, BN, 32), lambda i: (0, i, 0)),
            pl.BlockSpec((BN,), lambda i: (i,)),
            pl.BlockSpec((32, 64), lambda i: (0, 0)),
            pl.BlockSpec((64,), lambda i: (0,)),
            pl.BlockSpec((2, 64, 64), lambda i: (0, 0, 0)),
            pl.BlockSpec((64,), lambda i: (0,)),
            pl.BlockSpec((64, 64), lambda i: (0, 0)),
            pl.BlockSpec((64,), lambda i: (0,)),
            pl.BlockSpec((64, HOR * F_IN), lambda i: (0, 0)),
            pl.BlockSpec((HOR * F_IN,), lambda i: (0,)),
        ],
        out_specs=pl.BlockSpec((BN, HOR * F_IN), lambda i: (i, 0)),
        out_shape=jax.ShapeDtypeStruct((N_PAD, HOR * F_IN), jnp.float32),
    )(s2, x2d, dinv, w2, b2, tw2t, tb2, fw1, fb1, fw2, fb2)


def kernel(x, edge_index, W1, b1, tw1, tb1, W2, b2, tw2, tb2,
           fw1, fb1, fw2, fb2):
    n = x.shape[0]
    x12 = x[:, T_STEPS - 3:, :].reshape(n, 3 * F_IN)
    x12 = jnp.pad(x12, ((0, N_PAD - n), (0, 0)))

    pad_n = E_PAD - E_EDGES
    ar = jnp.arange(pad_n, dtype=jnp.int32)
    src_flat = jnp.concatenate([edge_index[0], ar % N_NODES])
    dstp = jnp.concatenate(
        [edge_index[1], N_NODES + (ar % (N_PAD - N_NODES))]).reshape(
        IDX_ROWS, 128)
    src2 = jnp.stack([src_flat, src_flat + N_PAD])

    tw1t = jnp.transpose(tw1, (2, 1, 0))
    tw2t = jnp.transpose(tw2, (2, 1, 0))[:2]

    deg2 = _deg_call(dstp).reshape(NC, N_PAD)
    xd16, dinv = _prep_call(deg2, x12)
    s1 = _s1_call(xd16, src_flat, dstp)
    x2d = _mid_call(s1, xd16, dinv, W1, b1, tw1t, tb1)
    tab2 = x2d.reshape(2 * N_PAD, 32)
    s2 = _s2_call(tab2, src2, dstp)
    out = _head_call(s2, x2d, dinv, W2, b2, tw2t, tb2, fw1, fb1, fw2, fb2)
    return out[:n].reshape(n, HOR, F_IN)


# speedup vs baseline: 1.3045x; 1.1474x over previous
import functools

import jax
import jax.numpy as jnp
from jax import lax
from jax.experimental import pallas as pl
from jax.experimental.pallas import tpu as pltpu
from jax.experimental.pallas import tpu_sc as plsc

N_NODES = 50000
N_PAD = 50176
T_STEPS = 12
F_IN = 4
HOR = 3
E_EDGES = 800000
E_PAD = 819200
IDX_ROWS = E_PAD // 128
NC = 2
NS = 16
ROWS_PER_TILE = N_PAD // NS
KB = 8
KB_S2 = 4

BN = 7168
NBLK = N_PAD // BN

_mesh = plsc.VectorSubcoreMesh(
    core_axis_name="c", subcore_axis_name="s", num_cores=NC, num_subcores=NS)


def _al8(v):
    return pl.multiple_of(v, 8)


def _zero_fill_1d(buf, n):
    z = jnp.zeros((16,), jnp.float32)

    def step(i, _):
        buf[pl.ds(i * 16, 16)] = z
        return 0

    lax.fori_loop(0, n // 16, step, 0)


def _zero_fill_2d(buf, rows, cols):
    z = jnp.zeros((16,), jnp.float32)

    def step(i, _):
        for c0 in range(0, cols, 16):
            buf[i, pl.ds(c0, 16)] = z
        return 0

    lax.fori_loop(0, rows, step, 0)


def _deg_body(dst_hbm, out_hbm, deg_sh, zbuf, ones_v, idx_v, ssem):
    cid = lax.axis_index("c")
    sid = lax.axis_index("s")
    _zero_fill_1d(zbuf, ROWS_PER_TILE)
    one = jnp.ones((16,), jnp.float32)
    for i in range(8):
        ones_v[pl.ds(i * 16, 16)] = one
    lo = _al8(sid * ROWS_PER_TILE)
    pltpu.sync_copy(zbuf, deg_sh.at[pl.ds(lo, ROWS_PER_TILE)])
    plsc.subcore_barrier()

    w = cid * NS + sid
    n_rows = IDX_ROWS // (NC * NS)

    def blk(b, _):
        r0 = _al8(w * n_rows + b * KB)
        pltpu.sync_copy(dst_hbm.at[pl.ds(r0, KB)], idx_v)
        for j in range(KB):
            pltpu.async_copy(ones_v, deg_sh.at[idx_v.at[j]], ssem, add=True)
        for j in range(KB):
            pltpu.make_async_copy(ones_v, deg_sh.at[idx_v.at[j]],
                                  ssem).wait()
        return 0

    lax.fori_loop(0, n_rows // KB, blk, 0)
    plsc.subcore_barrier()
    pltpu.sync_copy(deg_sh.at[pl.ds(lo, ROWS_PER_TILE)], zbuf)
    pltpu.sync_copy(zbuf,
                    out_hbm.at[pl.ds(_al8(cid * N_PAD + lo), ROWS_PER_TILE)])


_deg_call = functools.partial(
    pl.kernel,
    _deg_body,
    out_type=jax.ShapeDtypeStruct((NC * N_PAD,), jnp.float32),
    mesh=_mesh,
    scratch_types=[
        pltpu.VMEM_SHARED((N_PAD,), jnp.float32),
        pltpu.VMEM((ROWS_PER_TILE,), jnp.float32),
        pltpu.VMEM((128,), jnp.float32),
        pltpu.VMEM((KB, 128), jnp.int32),
        pltpu.SemaphoreType.DMA,
    ],
    compiler_params=pltpu.CompilerParams(use_tc_tiling_on_sc=False),
)()


def _s1_body(tab_hbm, src_hbm, dst_hbm, out_hbm,
             acc_sh, zbuf, srcv, dstv, rows_v, sem, ssem):
    cid = lax.axis_index("c")
    sid = lax.axis_index("s")
    _zero_fill_2d(zbuf, ROWS_PER_TILE // 4, 16)
    lo = _al8(sid * ROWS_PER_TILE)
    for q in range(4):
        pltpu.sync_copy(zbuf, acc_sh.at[pl.ds(
            _al8(lo + q * (ROWS_PER_TILE // 4)), ROWS_PER_TILE // 4)])
    plsc.subcore_barrier()

    w = cid * NS + sid
    n_rows = IDX_ROWS // (NC * NS)

    def blk(b, _):
        r0 = _al8(w * n_rows + b * KB)
        pltpu.sync_copy(src_hbm.at[pl.ds(_al8(r0 * 128), KB * 128)], srcv)
        pltpu.sync_copy(dst_hbm.at[pl.ds(r0, KB)], dstv)
        pltpu.async_copy(tab_hbm.at[srcv], rows_v, sem).wait()
        for j in range(KB):
            pltpu.async_copy(rows_v.at[pl.ds(j * 128, 128)],
                             acc_sh.at[dstv.at[j]], ssem, add=True)
        for j in range(KB):
            pltpu.make_async_copy(rows_v.at[pl.ds(j * 128, 128)],
                                  acc_sh.at[dstv.at[j]], ssem).wait()
        return 0

    lax.fori_loop(0, n_rows // KB, blk, 0)
    plsc.subcore_barrier()
    for q in range(4):
        qlo = _al8(lo + q * (ROWS_PER_TILE // 4))
        pltpu.sync_copy(acc_sh.at[pl.ds(qlo, ROWS_PER_TILE // 4)], zbuf)
        pltpu.sync_copy(zbuf, out_hbm.at[cid, pl.ds(qlo, ROWS_PER_TILE // 4)])


_s1_call = functools.partial(
    pl.kernel,
    _s1_body,
    out_type=jax.ShapeDtypeStruct((NC, N_PAD, 16), jnp.float32),
    mesh=_mesh,
    scratch_types=[
        pltpu.VMEM_SHARED((N_PAD, 16), jnp.float32),
        pltpu.VMEM((ROWS_PER_TILE // 4, 16), jnp.float32),
        pltpu.VMEM((KB * 128,), jnp.int32),
        pltpu.VMEM((KB, 128), jnp.int32),
        pltpu.VMEM((KB * 128, 16), jnp.float32),
        pltpu.SemaphoreType.DMA,
        pltpu.SemaphoreType.DMA,
    ],
    compiler_params=pltpu.CompilerParams(use_tc_tiling_on_sc=False),
)()


def _s2_body(tab_hbm, src2_hbm, dst_hbm, out_hbm,
             acc_sh, zbuf, srcv, dstv, srcw, dstw, rows_v,
             sem, ssem, isem0, isem1):
    cid = lax.axis_index("c")
    sid = lax.axis_index("s")
    _zero_fill_2d(zbuf, 112, 32)
    lo = _al8(sid * ROWS_PER_TILE)
    for q in range(28):
        pltpu.sync_copy(zbuf, acc_sh.at[pl.ds(_al8(lo + q * 112), 112)])
    plsc.subcore_barrier()

    n_rows = IDX_ROWS // NS
    n_blocks = n_rows // KB_S2

    def _r0(b):
        return pl.multiple_of(sid * n_rows + b * KB_S2, 4)

    def _fire_idx(b, sv, dv, isem):
        r0 = _r0(b)
        pltpu.async_copy(
            src2_hbm.at[cid, pl.ds(_al8(r0 * 128), KB_S2 * 128)], sv, isem)
        pltpu.async_copy(dst_hbm.at[pl.ds(r0, KB_S2)], dv, isem)

    def _wait_idx(b, sv, dv, isem):
        r0 = _r0(b)
        pltpu.make_async_copy(
            src2_hbm.at[cid, pl.ds(_al8(r0 * 128), KB_S2 * 128)],
            sv, isem).wait()
        pltpu.make_async_copy(dst_hbm.at[pl.ds(r0, KB_S2)], dv, isem).wait()

    idx_bufs = ((srcv, dstv, isem0), (srcw, dstw, isem1))
    _fire_idx(0, *idx_bufs[0])

    def pair(p, _):
        for h in range(2):
            b = 2 * p + h
            sv, dv, isem = idx_bufs[h]
            nsv, ndv, nisem = idx_bufs[1 - h]
            _wait_idx(b, sv, dv, isem)
            bn = lax.rem(b + 1, n_blocks)
            _fire_idx(bn, nsv, ndv, nisem)
            pltpu.async_copy(tab_hbm.at[sv], rows_v, sem).wait()
            for j in range(KB_S2):
                pltpu.async_copy(rows_v.at[pl.ds(j * 128, 128)],
                                 acc_sh.at[dv.at[j]], ssem, add=True)
            for j in range(KB_S2):
                pltpu.make_async_copy(rows_v.at[pl.ds(j * 128, 128)],
                                      acc_sh.at[dv.at[j]], ssem).wait()
        return 0

    lax.fori_loop(0, n_blocks // 2, pair, 0)
    pltpu.make_async_copy(
        src2_hbm.at[cid, pl.ds(_al8(_r0(0) * 128), KB_S2 * 128)],
        srcv, isem0).wait()
    pltpu.make_async_copy(dst_hbm.at[pl.ds(_r0(0), KB_S2)], dstv,
                          isem0).wait()
    plsc.subcore_barrier()
    for q in range(28):
        qlo = _al8(lo + q * 112)
        pltpu.sync_copy(acc_sh.at[pl.ds(qlo, 112)], zbuf)
        pltpu.sync_copy(zbuf, out_hbm.at[cid, pl.ds(qlo, 112)])


_s2_call = functools.partial(
    pl.kernel,
    _s2_body,
    out_type=jax.ShapeDtypeStruct((NC, N_PAD, 32), jnp.float32),
    mesh=_mesh,
    scratch_types=[
        pltpu.VMEM_SHARED((N_PAD, 32), jnp.float32),
        pltpu.VMEM((112, 32), jnp.float32),
        pltpu.VMEM((KB_S2 * 128,), jnp.int32),
        pltpu.VMEM((KB_S2, 128), jnp.int32),
        pltpu.VMEM((KB_S2 * 128,), jnp.int32),
        pltpu.VMEM((KB_S2, 128), jnp.int32),
        pltpu.VMEM((KB_S2 * 128, 32), jnp.float32),
        pltpu.SemaphoreType.DMA,
        pltpu.SemaphoreType.DMA,
        pltpu.SemaphoreType.DMA,
        pltpu.SemaphoreType.DMA,
    ],
    compiler_params=pltpu.CompilerParams(use_tc_tiling_on_sc=False),
)()


def _prep_body(deg_ref, x12_ref, xd_ref, dinv_ref):
    deg = deg_ref[0, :] + deg_ref[1, :] + 1.0
    dinv = lax.rsqrt(deg)
    dinv_ref[...] = dinv
    xd12 = x12_ref[...] * dinv[:, None]
    xd_ref[...] = jnp.concatenate(
        [xd12, jnp.zeros((BN, 4), jnp.float32)], axis=1)


def _prep_call(deg2, x12):
    return pl.pallas_call(
        _prep_body,
        grid=(NBLK,),
        in_specs=[
            pl.BlockSpec((NC, BN), lambda i: (0, i)),
            pl.BlockSpec((BN, 12), lambda i: (i, 0)),
        ],
        out_specs=[
            pl.BlockSpec((BN, 16), lambda i: (i, 0)),
            pl.BlockSpec((BN,), lambda i: (i,)),
        ],
        out_shape=[
            jax.ShapeDtypeStruct((N_PAD, 16), jnp.float32),
            jax.ShapeDtypeStruct((N_PAD,), jnp.float32),
        ],
    )(deg2, x12)


def _mid_body(s1_ref, xd_ref, dinv_ref, w1_ref, b1_ref, k_ref, tb1_ref,
              x2d_ref):
    dinv = dinv_ref[...][:, None]
    y = (s1_ref[0] + s1_ref[1] + xd_ref[...]) * dinv
    w1 = w1_ref[...]
    b1 = b1_ref[...]

    def gcn(t):
        return jnp.maximum(
            jnp.dot(y[:, 4 * t:4 * t + 4], w1,
                    preferred_element_type=jnp.float32) + b1, 0.0)

    g9, g10, g11 = gcn(0), gcn(1), gcn(2)
    k0, k1, k2 = k_ref[0], k_ref[1], k_ref[2]
    tb1 = tb1_ref[...]
    o10 = jnp.maximum(
        jnp.dot(g9, k0, preferred_element_type=jnp.float32)
        + jnp.dot(g10, k1, preferred_element_type=jnp.float32)
        + jnp.dot(g11, k2, preferred_element_type=jnp.float32) + tb1, 0.0)
    o11 = jnp.maximum(
        jnp.dot(g10, k0, preferred_element_type=jnp.float32)
        + jnp.dot(g11, k1, preferred_element_type=jnp.float32) + tb1, 0.0)
    x2d_ref[0] = o10 * dinv
    x2d_ref[1] = o11 * dinv


def _mid_call(s1, xd16, dinv, w1, b1, tw1t, tb1):
    return pl.pallas_call(
        _mid_body,
        grid=(NBLK,),
        in_specs=[
            pl.BlockSpec((NC, BN, 16), lambda i: (0, i, 0)),
            pl.BlockSpec((BN, 16), lambda i: (i, 0)),
            pl.BlockSpec((BN,), lambda i: (i,)),
            pl.BlockSpec((F_IN, 32), lambda i: (0, 0)),
            pl.BlockSpec((32,), lambda i: (0,)),
            pl.BlockSpec((3, 32, 32), lambda i: (0, 0, 0)),
            pl.BlockSpec((32,), lambda i: (0,)),
        ],
        out_specs=pl.BlockSpec((2, BN, 32), lambda i: (0, i, 0)),
        out_shape=jax.ShapeDtypeStruct((2, N_PAD, 32), jnp.float32),
    )(s1, xd16, dinv, w1, b1, tw1t, tb1)


def _head_body(s2_ref, x2d_ref, dinv_ref, w2_ref, b2_ref, q_ref, tb2_ref,
               fw1_ref, fb1_ref, fw2_ref, fb2_ref, out_ref):
    dinv = dinv_ref[...][:, None]
    y10 = (s2_ref[0] + x2d_ref[0]) * dinv
    y11 = (s2_ref[1] + x2d_ref[1]) * dinv
    w2 = w2_ref[...]
    b2 = b2_ref[...]
    g10 = jnp.maximum(
        jnp.dot(y10, w2, preferred_element_type=jnp.float32) + b2, 0.0)
    g11 = jnp.maximum(
        jnp.dot(y11, w2, preferred_element_type=jnp.float32) + b2, 0.0)
    h = jnp.maximum(
        jnp.dot(g10, q_ref[0], preferred_element_type=jnp.float32)
        + jnp.dot(g11, q_ref[1], preferred_element_type=jnp.float32)
        + tb2_ref[...], 0.0)
    f = jnp.maximum(
        jnp.dot(h, fw1_ref[...], preferred_element_type=jnp.float32)
        + fb1_ref[...], 0.0)
    p = jnp.dot(f, fw2_ref[...], preferred_element_type=jnp.float32) \
        + fb2_ref[...]
    out_ref[...] = jnp.maximum(p, 0.0) + jnp.log1p(jnp.exp(-jnp.abs(p)))


def _head_call(s2, x2d, dinv, w2, b2, tw2t, tb2, fw1, fb1, fw2, fb2):
    return pl.pallas_call(
        _head_body,
        grid=(NBLK,),
        in_specs=[
            pl.BlockSpec((NC, BN, 32), lambda i: (0, i, 0)),
            pl.BlockSpec((2, BN, 32), lambda i: (0, i, 0)),
            pl.BlockSpec((BN,), lambda i: (i,)),
            pl.BlockSpec((32, 64), lambda i: (0, 0)),
            pl.BlockSpec((64,), lambda i: (0,)),
            pl.BlockSpec((2, 64, 64), lambda i: (0, 0, 0)),
            pl.BlockSpec((64,), lambda i: (0,)),
            pl.BlockSpec((64, 64), lambda i: (0, 0)),
            pl.BlockSpec((64,), lambda i: (0,)),
            pl.BlockSpec((64, HOR * F_IN), lambda i: (0, 0)),
            pl.BlockSpec((HOR * F_IN,), lambda i: (0,)),
        ],
        out_specs=pl.BlockSpec((BN, HOR * F_IN), lambda i: (i, 0)),
        out_shape=jax.ShapeDtypeStruct((N_PAD, HOR * F_IN), jnp.float32),
    )(s2, x2d, dinv, w2, b2, tw2t, tb2, fw1, fb1, fw2, fb2)


def kernel(x, edge_index, W1, b1, tw1, tb1, W2, b2, tw2, tb2,
           fw1, fb1, fw2, fb2):
    n = x.shape[0]
    x12 = x[:, T_STEPS - 3:, :].reshape(n, 3 * F_IN)
    x12 = jnp.pad(x12, ((0, N_PAD - n), (0, 0)))

    pad_n = E_PAD - E_EDGES
    ar = jnp.arange(pad_n, dtype=jnp.int32)
    src_flat = jnp.concatenate([edge_index[0], ar % N_NODES])
    dstp = jnp.concatenate(
        [edge_index[1], N_NODES + (ar % (N_PAD - N_NODES))]).reshape(
        IDX_ROWS, 128)
    src2 = jnp.stack([src_flat, src_flat + N_PAD])

    tw1t = jnp.transpose(tw1, (2, 1, 0))
    tw2t = jnp.transpose(tw2, (2, 1, 0))[:2]

    deg2 = _deg_call(dstp).reshape(NC, N_PAD)
    xd16, dinv = _prep_call(deg2, x12)
    s1 = _s1_call(xd16, src_flat, dstp)
    x2d = _mid_call(s1, xd16, dinv, W1, b1, tw1t, tb1)
    tab2 = x2d.reshape(2 * N_PAD, 32)
    s2 = _s2_call(tab2, src2, dstp)
    out = _head_call(s2, x2d, dinv, W2, b2, tw2t, tb2, fw1, fb1, fw2, fb2)
    return out[:n].reshape(n, HOR, F_IN)


# speedup vs baseline: 1.3676x; 1.0483x over previous
import functools

import jax
import jax.numpy as jnp
from jax import lax
from jax.experimental import pallas as pl
from jax.experimental.pallas import tpu as pltpu
from jax.experimental.pallas import tpu_sc as plsc

N_NODES = 50000
N_PAD = 50176
T_STEPS = 12
F_IN = 4
HOR = 3
E_EDGES = 800000
E_PAD = 819200
IDX_ROWS = E_PAD // 128
NC = 2
NS = 16
ROWS_PER_TILE = N_PAD // NS
KB = 10
KB_S2 = 4

BN = 7168
NBLK = N_PAD // BN

_mesh = plsc.VectorSubcoreMesh(
    core_axis_name="c", subcore_axis_name="s", num_cores=NC, num_subcores=NS)


def _al8(v):
    return pl.multiple_of(v, 8)


def _zero_fill_1d(buf, n):
    z = jnp.zeros((16,), jnp.float32)

    def step(i, _):
        buf[pl.ds(i * 16, 16)] = z
        return 0

    lax.fori_loop(0, n // 16, step, 0)


def _zero_fill_2d(buf, rows, cols):
    z = jnp.zeros((16,), jnp.float32)

    def step(i, _):
        for c0 in range(0, cols, 16):
            buf[i, pl.ds(c0, 16)] = z
        return 0

    lax.fori_loop(0, rows, step, 0)


def _deg_body(dst_hbm, out_hbm, deg_sh, zbuf, ones_v, idx_v, idx_w,
              ssem, isem0, isem1):
    cid = lax.axis_index("c")
    sid = lax.axis_index("s")
    _zero_fill_1d(zbuf, ROWS_PER_TILE)
    one = jnp.ones((16,), jnp.float32)
    for i in range(8):
        ones_v[pl.ds(i * 16, 16)] = one
    lo = _al8(sid * ROWS_PER_TILE)
    pltpu.sync_copy(zbuf, deg_sh.at[pl.ds(lo, ROWS_PER_TILE)])
    plsc.subcore_barrier()

    w = cid * NS + sid
    n_rows = IDX_ROWS // (NC * NS)
    n_blocks = n_rows // KB

    def _r0(b):
        return pl.multiple_of(w * n_rows + b * KB, 2)

    idx_bufs = ((idx_v, isem0), (idx_w, isem1))
    pltpu.async_copy(dst_hbm.at[pl.ds(_r0(0), KB)], idx_v, isem0)

    def pair(p, _):
        for h in range(2):
            b = 2 * p + h
            iv, isem = idx_bufs[h]
            nv, nisem = idx_bufs[1 - h]
            pltpu.make_async_copy(dst_hbm.at[pl.ds(_r0(b), KB)], iv,
                                  isem).wait()
            bn = lax.rem(b + 1, n_blocks)
            pltpu.async_copy(dst_hbm.at[pl.ds(_r0(bn), KB)], nv, nisem)
            for j in range(KB):
                pltpu.async_copy(ones_v, deg_sh.at[iv.at[j]], ssem, add=True)
            for j in range(KB):
                pltpu.make_async_copy(ones_v, deg_sh.at[iv.at[j]],
                                      ssem).wait()
        return 0

    lax.fori_loop(0, n_blocks // 2, pair, 0)
    pltpu.make_async_copy(dst_hbm.at[pl.ds(_r0(0), KB)], idx_v,
                          isem0).wait()
    plsc.subcore_barrier()
    pltpu.sync_copy(deg_sh.at[pl.ds(lo, ROWS_PER_TILE)], zbuf)
    pltpu.sync_copy(zbuf,
                    out_hbm.at[pl.ds(_al8(cid * N_PAD + lo), ROWS_PER_TILE)])


_deg_call = functools.partial(
    pl.kernel,
    _deg_body,
    out_type=jax.ShapeDtypeStruct((NC * N_PAD,), jnp.float32),
    mesh=_mesh,
    scratch_types=[
        pltpu.VMEM_SHARED((N_PAD,), jnp.float32),
        pltpu.VMEM((ROWS_PER_TILE,), jnp.float32),
        pltpu.VMEM((128,), jnp.float32),
        pltpu.VMEM((KB, 128), jnp.int32),
        pltpu.VMEM((KB, 128), jnp.int32),
        pltpu.SemaphoreType.DMA,
        pltpu.SemaphoreType.DMA,
        pltpu.SemaphoreType.DMA,
    ],
    compiler_params=pltpu.CompilerParams(use_tc_tiling_on_sc=False),
)()


def _s1_body(tab_hbm, src_hbm, dst_hbm, out_hbm,
             acc_sh, zbuf, srcv, dstv, srcw, dstw, rows_v,
             sem, ssem, isem0, isem1):
    cid = lax.axis_index("c")
    sid = lax.axis_index("s")
    _zero_fill_2d(zbuf, ROWS_PER_TILE // 4, 16)
    lo = _al8(sid * ROWS_PER_TILE)
    for q in range(4):
        pltpu.sync_copy(zbuf, acc_sh.at[pl.ds(
            _al8(lo + q * (ROWS_PER_TILE // 4)), ROWS_PER_TILE // 4)])
    plsc.subcore_barrier()

    w = cid * NS + sid
    n_rows = IDX_ROWS // (NC * NS)
    n_blocks = n_rows // KB

    def _r0(b):
        return pl.multiple_of(w * n_rows + b * KB, 2)

    def _fire_idx(b, sv, dv, isem):
        r0 = _r0(b)
        pltpu.async_copy(src_hbm.at[pl.ds(_al8(r0 * 128), KB * 128)],
                         sv, isem)
        pltpu.async_copy(dst_hbm.at[pl.ds(r0, KB)], dv, isem)

    def _wait_idx(b, sv, dv, isem):
        r0 = _r0(b)
        pltpu.make_async_copy(src_hbm.at[pl.ds(_al8(r0 * 128), KB * 128)],
                              sv, isem).wait()
        pltpu.make_async_copy(dst_hbm.at[pl.ds(r0, KB)], dv, isem).wait()

    idx_bufs = ((srcv, dstv, isem0), (srcw, dstw, isem1))
    _fire_idx(0, *idx_bufs[0])

    def pair(p, _):
        for h in range(2):
            b = 2 * p + h
            sv, dv, isem = idx_bufs[h]
            nsv, ndv, nisem = idx_bufs[1 - h]
            _wait_idx(b, sv, dv, isem)
            bn = lax.rem(b + 1, n_blocks)
            _fire_idx(bn, nsv, ndv, nisem)
            pltpu.async_copy(tab_hbm.at[sv], rows_v, sem).wait()
            for j in range(KB):
                pltpu.async_copy(rows_v.at[pl.ds(j * 128, 128)],
                                 acc_sh.at[dv.at[j]], ssem, add=True)
            for j in range(KB):
                pltpu.make_async_copy(rows_v.at[pl.ds(j * 128, 128)],
                                      acc_sh.at[dv.at[j]], ssem).wait()
        return 0

    lax.fori_loop(0, n_blocks // 2, pair, 0)
    _wait_idx(0, *idx_bufs[0])
    plsc.subcore_barrier()
    for q in range(4):
        qlo = _al8(lo + q * (ROWS_PER_TILE // 4))
        pltpu.sync_copy(acc_sh.at[pl.ds(qlo, ROWS_PER_TILE // 4)], zbuf)
        pltpu.sync_copy(zbuf, out_hbm.at[cid, pl.ds(qlo, ROWS_PER_TILE // 4)])


_s1_call = functools.partial(
    pl.kernel,
    _s1_body,
    out_type=jax.ShapeDtypeStruct((NC, N_PAD, 16), jnp.float32),
    mesh=_mesh,
    scratch_types=[
        pltpu.VMEM_SHARED((N_PAD, 16), jnp.float32),
        pltpu.VMEM((ROWS_PER_TILE // 4, 16), jnp.float32),
        pltpu.VMEM((KB * 128,), jnp.int32),
        pltpu.VMEM((KB, 128), jnp.int32),
        pltpu.VMEM((KB * 128,), jnp.int32),
        pltpu.VMEM((KB, 128), jnp.int32),
        pltpu.VMEM((KB * 128, 16), jnp.float32),
        pltpu.SemaphoreType.DMA,
        pltpu.SemaphoreType.DMA,
        pltpu.SemaphoreType.DMA,
        pltpu.SemaphoreType.DMA,
    ],
    compiler_params=pltpu.CompilerParams(use_tc_tiling_on_sc=False),
)()


def _s2_body(tab_hbm, src2_hbm, dst_hbm, out_hbm,
             acc_sh, zbuf, srcv, dstv, srcw, dstw, rows_v,
             sem, ssem, isem0, isem1):
    cid = lax.axis_index("c")
    sid = lax.axis_index("s")
    _zero_fill_2d(zbuf, 112, 32)
    lo = _al8(sid * ROWS_PER_TILE)
    for q in range(28):
        pltpu.sync_copy(zbuf, acc_sh.at[pl.ds(_al8(lo + q * 112), 112)])
    plsc.subcore_barrier()

    n_rows = IDX_ROWS // NS
    n_blocks = n_rows // KB_S2

    def _r0(b):
        return pl.multiple_of(sid * n_rows + b * KB_S2, 4)

    def _fire_idx(b, sv, dv, isem):
        r0 = _r0(b)
        pltpu.async_copy(
            src2_hbm.at[cid, pl.ds(_al8(r0 * 128), KB_S2 * 128)], sv, isem)
        pltpu.async_copy(dst_hbm.at[pl.ds(r0, KB_S2)], dv, isem)

    def _wait_idx(b, sv, dv, isem):
        r0 = _r0(b)
        pltpu.make_async_copy(
            src2_hbm.at[cid, pl.ds(_al8(r0 * 128), KB_S2 * 128)],
            sv, isem).wait()
        pltpu.make_async_copy(dst_hbm.at[pl.ds(r0, KB_S2)], dv, isem).wait()

    idx_bufs = ((srcv, dstv, isem0), (srcw, dstw, isem1))
    _fire_idx(0, *idx_bufs[0])

    def pair(p, _):
        for h in range(2):
            b = 2 * p + h
            sv, dv, isem = idx_bufs[h]
            nsv, ndv, nisem = idx_bufs[1 - h]
            _wait_idx(b, sv, dv, isem)
            bn = lax.rem(b + 1, n_blocks)
            _fire_idx(bn, nsv, ndv, nisem)
            pltpu.async_copy(tab_hbm.at[sv], rows_v, sem).wait()
            for j in range(KB_S2):
                pltpu.async_copy(rows_v.at[pl.ds(j * 128, 128)],
                                 acc_sh.at[dv.at[j]], ssem, add=True)
            for j in range(KB_S2):
                pltpu.make_async_copy(rows_v.at[pl.ds(j * 128, 128)],
                                      acc_sh.at[dv.at[j]], ssem).wait()
        return 0

    lax.fori_loop(0, n_blocks // 2, pair, 0)
    pltpu.make_async_copy(
        src2_hbm.at[cid, pl.ds(_al8(_r0(0) * 128), KB_S2 * 128)],
        srcv, isem0).wait()
    pltpu.make_async_copy(dst_hbm.at[pl.ds(_r0(0), KB_S2)], dstv,
                          isem0).wait()
    plsc.subcore_barrier()
    for q in range(28):
        qlo = _al8(lo + q * 112)
        pltpu.sync_copy(acc_sh.at[pl.ds(qlo, 112)], zbuf)
        pltpu.sync_copy(zbuf, out_hbm.at[cid, pl.ds(qlo, 112)])


_s2_call = functools.partial(
    pl.kernel,
    _s2_body,
    out_type=jax.ShapeDtypeStruct((NC, N_PAD, 32), jnp.float32),
    mesh=_mesh,
    scratch_types=[
        pltpu.VMEM_SHARED((N_PAD, 32), jnp.float32),
        pltpu.VMEM((112, 32), jnp.float32),
        pltpu.VMEM((KB_S2 * 128,), jnp.int32),
        pltpu.VMEM((KB_S2, 128), jnp.int32),
        pltpu.VMEM((KB_S2 * 128,), jnp.int32),
        pltpu.VMEM((KB_S2, 128), jnp.int32),
        pltpu.VMEM((KB_S2 * 128, 32), jnp.float32),
        pltpu.SemaphoreType.DMA,
        pltpu.SemaphoreType.DMA,
        pltpu.SemaphoreType.DMA,
        pltpu.SemaphoreType.DMA,
    ],
    compiler_params=pltpu.CompilerParams(use_tc_tiling_on_sc=False),
)()


def _prep_body(deg_ref, x12_ref, xd_ref, dinv_ref):
    deg = deg_ref[0, :] + deg_ref[1, :] + 1.0
    dinv = lax.rsqrt(deg)
    dinv_ref[...] = dinv
    xd12 = x12_ref[...] * dinv[:, None]
    xd_ref[...] = jnp.concatenate(
        [xd12, jnp.zeros((BN, 4), jnp.float32)], axis=1)


def _prep_call(deg2, x12):
    return pl.pallas_call(
        _prep_body,
        grid=(NBLK,),
        in_specs=[
            pl.BlockSpec((NC, BN), lambda i: (0, i)),
            pl.BlockSpec((BN, 12), lambda i: (i, 0)),
        ],
        out_specs=[
            pl.BlockSpec((BN, 16), lambda i: (i, 0)),
            pl.BlockSpec((BN,), lambda i: (i,)),
        ],
        out_shape=[
            jax.ShapeDtypeStruct((N_PAD, 16), jnp.float32),
            jax.ShapeDtypeStruct((N_PAD,), jnp.float32),
        ],
    )(deg2, x12)


def _mid_body(s1_ref, xd_ref, dinv_ref, w1_ref, b1_ref, k_ref, tb1_ref,
              x2d_ref):
    dinv = dinv_ref[...][:, None]
    y = (s1_ref[0] + s1_ref[1] + xd_ref[...]) * dinv
    w1 = w1_ref[...]
    b1 = b1_ref[...]

    def gcn(t):
        return jnp.maximum(
            jnp.dot(y[:, 4 * t:4 * t + 4], w1,
                    preferred_element_type=jnp.float32) + b1, 0.0)

    g9, g10, g11 = gcn(0), gcn(1), gcn(2)
    k0, k1, k2 = k_ref[0], k_ref[1], k_ref[2]
    tb1 = tb1_ref[...]
    o10 = jnp.maximum(
        jnp.dot(g9, k0, preferred_element_type=jnp.float32)
        + jnp.dot(g10, k1, preferred_element_type=jnp.float32)
        + jnp.dot(g11, k2, preferred_element_type=jnp.float32) + tb1, 0.0)
    o11 = jnp.maximum(
        jnp.dot(g10, k0, preferred_element_type=jnp.float32)
        + jnp.dot(g11, k1, preferred_element_type=jnp.float32) + tb1, 0.0)
    x2d_ref[0] = o10 * dinv
    x2d_ref[1] = o11 * dinv


def _mid_call(s1, xd16, dinv, w1, b1, tw1t, tb1):
    return pl.pallas_call(
        _mid_body,
        grid=(NBLK,),
        in_specs=[
            pl.BlockSpec((NC, BN, 16), lambda i: (0, i, 0)),
            pl.BlockSpec((BN, 16), lambda i: (i, 0)),
            pl.BlockSpec((BN,), lambda i: (i,)),
            pl.BlockSpec((F_IN, 32), lambda i: (0, 0)),
            pl.BlockSpec((32,), lambda i: (0,)),
            pl.BlockSpec((3, 32, 32), lambda i: (0, 0, 0)),
            pl.BlockSpec((32,), lambda i: (0,)),
        ],
        out_specs=pl.BlockSpec((2, BN, 32), lambda i: (0, i, 0)),
        out_shape=jax.ShapeDtypeStruct((2, N_PAD, 32), jnp.float32),
    )(s1, xd16, dinv, w1, b1, tw1t, tb1)


def _head_body(s2_ref, x2d_ref, dinv_ref, w2_ref, b2_ref, q_ref, tb2_ref,
               fw1_ref, fb1_ref, fw2_ref, fb2_ref, out_ref):
    dinv = dinv_ref[...][:, None]
    y10 = (s2_ref[0] + x2d_ref[0]) * dinv
    y11 = (s2_ref[1] + x2d_ref[1]) * dinv
    w2 = w2_ref[...]
    b2 = b2_ref[...]
    g10 = jnp.maximum(
        jnp.dot(y10, w2, preferred_element_type=jnp.float32) + b2, 0.0)
    g11 = jnp.maximum(
        jnp.dot(y11, w2, preferred_element_type=jnp.float32) + b2, 0.0)
    h = jnp.maximum(
        jnp.dot(g10, q_ref[0], preferred_element_type=jnp.float32)
        + jnp.dot(g11, q_ref[1], preferred_element_type=jnp.float32)
        + tb2_ref[...], 0.0)
    f = jnp.maximum(
        jnp.dot(h, fw1_ref[...], preferred_element_type=jnp.float32)
        + fb1_ref[...], 0.0)
    p = jnp.dot(f, fw2_ref[...], preferred_element_type=jnp.float32) \
        + fb2_ref[...]
    out_ref[...] = jnp.maximum(p, 0.0) + jnp.log1p(jnp.exp(-jnp.abs(p)))


def _head_call(s2, x2d, dinv, w2, b2, tw2t, tb2, fw1, fb1, fw2, fb2):
    return pl.pallas_call(
        _head_body,
        grid=(NBLK,),
        in_specs=[
            pl.BlockSpec((NC, BN, 32), lambda i: (0, i, 0)),
            pl.BlockSpec((2, BN, 32), lambda i: (0, i, 0)),
            pl.BlockSpec((BN,), lambda i: (i,)),
            pl.BlockSpec((32, 64), lambda i: (0, 0)),
            pl.BlockSpec((64,), lambda i: (0,)),
            pl.BlockSpec((2, 64, 64), lambda i: (0, 0, 0)),
            pl.BlockSpec((64,), lambda i: (0,)),
            pl.BlockSpec((64, 64), lambda i: (0, 0)),
            pl.BlockSpec((64,), lambda i: (0,)),
            pl.BlockSpec((64, HOR * F_IN), lambda i: (0, 0)),
            pl.BlockSpec((HOR * F_IN,), lambda i: (0,)),
        ],
        out_specs=pl.BlockSpec((BN, HOR * F_IN), lambda i: (i, 0)),
        out_shape=jax.ShapeDtypeStruct((N_PAD, HOR * F_IN), jnp.float32),
    )(s2, x2d, dinv, w2, b2, tw2t, tb2, fw1, fb1, fw2, fb2)


def kernel(x, edge_index, W1, b1, tw1, tb1, W2, b2, tw2, tb2,
           fw1, fb1, fw2, fb2):
    n = x.shape[0]
    x12 = x[:, T_STEPS - 3:, :].reshape(n, 3 * F_IN)
    x12 = jnp.pad(x12, ((0, N_PAD - n), (0, 0)))

    pad_n = E_PAD - E_EDGES
    ar = jnp.arange(pad_n, dtype=jnp.int32)
    src_flat = jnp.concatenate([edge_index[0], ar % N_NODES])
    dstp = jnp.concatenate(
        [edge_index[1], N_NODES + (ar % (N_PAD - N_NODES))]).reshape(
        IDX_ROWS, 128)
    src2 = jnp.stack([src_flat, src_flat + N_PAD])

    tw1t = jnp.transpose(tw1, (2, 1, 0))
    tw2t = jnp.transpose(tw2, (2, 1, 0))[:2]

    deg2 = _deg_call(dstp).reshape(NC, N_PAD)
    xd16, dinv = _prep_call(deg2, x12)
    s1 = _s1_call(xd16, src_flat, dstp)
    x2d = _mid_call(s1, xd16, dinv, W1, b1, tw1t, tb1)
    tab2 = x2d.reshape(2 * N_PAD, 32)
    s2 = _s2_call(tab2, src2, dstp)
    out = _head_call(s2, x2d, dinv, W2, b2, tw2t, tb2, fw1, fb1, fw2, fb2)
    return out[:n].reshape(n, HOR, F_IN)


# speedup vs baseline: 1.3972x; 1.0216x over previous
import functools

import jax
import jax.numpy as jnp
from jax import lax
from jax.experimental import pallas as pl
from jax.experimental.pallas import tpu as pltpu
from jax.experimental.pallas import tpu_sc as plsc

N_NODES = 50000
N_PAD = 50176
T_STEPS = 12
F_IN = 4
HOR = 3
E_EDGES = 800000
E_PAD = 819200
IDX_ROWS = E_PAD // 128
NC = 2
NS = 16
ROWS_PER_TILE = N_PAD // NS
KB = 10
KB_S2 = 5

BN = 7168
NBLK = N_PAD // BN

_mesh = plsc.VectorSubcoreMesh(
    core_axis_name="c", subcore_axis_name="s", num_cores=NC, num_subcores=NS)


def _al8(v):
    return pl.multiple_of(v, 8)


def _zero_fill_1d(buf, n):
    z = jnp.zeros((16,), jnp.float32)

    def step(i, _):
        buf[pl.ds(i * 16, 16)] = z
        return 0

    lax.fori_loop(0, n // 16, step, 0)


def _zero_fill_2d(buf, rows, cols):
    z = jnp.zeros((16,), jnp.float32)

    def step(i, _):
        for c0 in range(0, cols, 16):
            buf[i, pl.ds(c0, 16)] = z
        return 0

    lax.fori_loop(0, rows, step, 0)


def _deg_body(dst_hbm, out_hbm, deg_sh, zbuf, ones_v, idx_v, idx_w,
              ssem, isem0, isem1):
    cid = lax.axis_index("c")
    sid = lax.axis_index("s")
    _zero_fill_1d(zbuf, ROWS_PER_TILE)
    one = jnp.ones((16,), jnp.float32)
    for i in range(8):
        ones_v[pl.ds(i * 16, 16)] = one
    lo = _al8(sid * ROWS_PER_TILE)
    pltpu.sync_copy(zbuf, deg_sh.at[pl.ds(lo, ROWS_PER_TILE)])
    plsc.subcore_barrier()

    w = cid * NS + sid
    n_rows = IDX_ROWS // (NC * NS)
    n_blocks = n_rows // KB

    def _r0(b):
        return pl.multiple_of(w * n_rows + b * KB, 2)

    idx_bufs = ((idx_v, isem0), (idx_w, isem1))
    pltpu.async_copy(dst_hbm.at[pl.ds(_r0(0), KB)], idx_v, isem0)

    def pair(p, _):
        for h in range(2):
            b = 2 * p + h
            iv, isem = idx_bufs[h]
            nv, nisem = idx_bufs[1 - h]
            pltpu.make_async_copy(dst_hbm.at[pl.ds(_r0(b), KB)], iv,
                                  isem).wait()
            bn = lax.rem(b + 1, n_blocks)
            pltpu.async_copy(dst_hbm.at[pl.ds(_r0(bn), KB)], nv, nisem)
            for j in range(KB):
                pltpu.async_copy(ones_v, deg_sh.at[iv.at[j]], ssem, add=True)
            for j in range(KB):
                pltpu.make_async_copy(ones_v, deg_sh.at[iv.at[j]],
                                      ssem).wait()
        return 0

    lax.fori_loop(0, n_blocks // 2, pair, 0)
    pltpu.make_async_copy(dst_hbm.at[pl.ds(_r0(0), KB)], idx_v,
                          isem0).wait()
    plsc.subcore_barrier()
    pltpu.sync_copy(deg_sh.at[pl.ds(lo, ROWS_PER_TILE)], zbuf)
    pltpu.sync_copy(zbuf,
                    out_hbm.at[pl.ds(_al8(cid * N_PAD + lo), ROWS_PER_TILE)])


_deg_call = functools.partial(
    pl.kernel,
    _deg_body,
    out_type=jax.ShapeDtypeStruct((NC * N_PAD,), jnp.float32),
    mesh=_mesh,
    scratch_types=[
        pltpu.VMEM_SHARED((N_PAD,), jnp.float32),
        pltpu.VMEM((ROWS_PER_TILE,), jnp.float32),
        pltpu.VMEM((128,), jnp.float32),
        pltpu.VMEM((KB, 128), jnp.int32),
        pltpu.VMEM((KB, 128), jnp.int32),
        pltpu.SemaphoreType.DMA,
        pltpu.SemaphoreType.DMA,
        pltpu.SemaphoreType.DMA,
    ],
    compiler_params=pltpu.CompilerParams(use_tc_tiling_on_sc=False),
)()


def _s1_body(tab_hbm, src_hbm, dst_hbm, out_hbm,
             acc_sh, zbuf, srcv, dstv, srcw, dstw, rows_v,
             sem, ssem, isem0, isem1):
    cid = lax.axis_index("c")
    sid = lax.axis_index("s")
    _zero_fill_2d(zbuf, ROWS_PER_TILE // 4, 16)
    lo = _al8(sid * ROWS_PER_TILE)
    for q in range(4):
        pltpu.sync_copy(zbuf, acc_sh.at[pl.ds(
            _al8(lo + q * (ROWS_PER_TILE // 4)), ROWS_PER_TILE // 4)])
    plsc.subcore_barrier()

    w = cid * NS + sid
    n_rows = IDX_ROWS // (NC * NS)
    n_blocks = n_rows // KB

    def _r0(b):
        return pl.multiple_of(w * n_rows + b * KB, 2)

    def _fire_idx(b, sv, dv, isem):
        r0 = _r0(b)
        pltpu.async_copy(src_hbm.at[pl.ds(_al8(r0 * 128), KB * 128)],
                         sv, isem)
        pltpu.async_copy(dst_hbm.at[pl.ds(r0, KB)], dv, isem)

    def _wait_idx(b, sv, dv, isem):
        r0 = _r0(b)
        pltpu.make_async_copy(src_hbm.at[pl.ds(_al8(r0 * 128), KB * 128)],
                              sv, isem).wait()
        pltpu.make_async_copy(dst_hbm.at[pl.ds(r0, KB)], dv, isem).wait()

    idx_bufs = ((srcv, dstv, isem0), (srcw, dstw, isem1))
    _fire_idx(0, *idx_bufs[0])

    def pair(p, _):
        for h in range(2):
            b = 2 * p + h
            sv, dv, isem = idx_bufs[h]
            nsv, ndv, nisem = idx_bufs[1 - h]
            _wait_idx(b, sv, dv, isem)
            bn = lax.rem(b + 1, n_blocks)
            _fire_idx(bn, nsv, ndv, nisem)
            pltpu.async_copy(tab_hbm.at[sv], rows_v, sem).wait()
            for j in range(KB):
                pltpu.async_copy(rows_v.at[pl.ds(j * 128, 128)],
                                 acc_sh.at[dv.at[j]], ssem, add=True)
            for j in range(KB):
                pltpu.make_async_copy(rows_v.at[pl.ds(j * 128, 128)],
                                      acc_sh.at[dv.at[j]], ssem).wait()
        return 0

    lax.fori_loop(0, n_blocks // 2, pair, 0)
    _wait_idx(0, *idx_bufs[0])
    plsc.subcore_barrier()
    for q in range(4):
        qlo = _al8(lo + q * (ROWS_PER_TILE // 4))
        pltpu.sync_copy(acc_sh.at[pl.ds(qlo, ROWS_PER_TILE // 4)], zbuf)
        pltpu.sync_copy(zbuf, out_hbm.at[cid, pl.ds(qlo, ROWS_PER_TILE // 4)])


_s1_call = functools.partial(
    pl.kernel,
    _s1_body,
    out_type=jax.ShapeDtypeStruct((NC, N_PAD, 16), jnp.float32),
    mesh=_mesh,
    scratch_types=[
        pltpu.VMEM_SHARED((N_PAD, 16), jnp.float32),
        pltpu.VMEM((ROWS_PER_TILE // 4, 16), jnp.float32),
        pltpu.VMEM((KB * 128,), jnp.int32),
        pltpu.VMEM((KB, 128), jnp.int32),
        pltpu.VMEM((KB * 128,), jnp.int32),
        pltpu.VMEM((KB, 128), jnp.int32),
        pltpu.VMEM((KB * 128, 16), jnp.float32),
        pltpu.SemaphoreType.DMA,
        pltpu.SemaphoreType.DMA,
        pltpu.SemaphoreType.DMA,
        pltpu.SemaphoreType.DMA,
    ],
    compiler_params=pltpu.CompilerParams(use_tc_tiling_on_sc=False),
)()


def _s2_body(tab_hbm, src2_hbm, dst_hbm, out_hbm,
             acc_sh, zbuf, srcv, dstv, srcw, dstw, rows_v,
             sem, ssem, isem0, isem1):
    cid = lax.axis_index("c")
    sid = lax.axis_index("s")
    _zero_fill_2d(zbuf, 112, 32)
    lo = _al8(sid * ROWS_PER_TILE)
    for q in range(28):
        pltpu.sync_copy(zbuf, acc_sh.at[pl.ds(_al8(lo + q * 112), 112)])
    plsc.subcore_barrier()

    n_rows = IDX_ROWS // NS
    n_blocks = n_rows // KB_S2

    def _r0(b):
        return sid * n_rows + b * KB_S2

    def _fire_idx(b, sv, dv, isem):
        r0 = _r0(b)
        pltpu.async_copy(
            src2_hbm.at[cid, pl.ds(_al8(r0 * 128), KB_S2 * 128)], sv, isem)
        pltpu.async_copy(dst_hbm.at[pl.ds(r0, KB_S2)], dv, isem)

    def _wait_idx(b, sv, dv, isem):
        r0 = _r0(b)
        pltpu.make_async_copy(
            src2_hbm.at[cid, pl.ds(_al8(r0 * 128), KB_S2 * 128)],
            sv, isem).wait()
        pltpu.make_async_copy(dst_hbm.at[pl.ds(r0, KB_S2)], dv, isem).wait()

    idx_bufs = ((srcv, dstv, isem0), (srcw, dstw, isem1))
    _fire_idx(0, *idx_bufs[0])

    def pair(p, _):
        for h in range(2):
            b = 2 * p + h
            sv, dv, isem = idx_bufs[h]
            nsv, ndv, nisem = idx_bufs[1 - h]
            _wait_idx(b, sv, dv, isem)
            bn = lax.rem(b + 1, n_blocks)
            _fire_idx(bn, nsv, ndv, nisem)
            pltpu.async_copy(tab_hbm.at[sv], rows_v, sem).wait()
            for j in range(KB_S2):
                pltpu.async_copy(rows_v.at[pl.ds(j * 128, 128)],
                                 acc_sh.at[dv.at[j]], ssem, add=True)
            for j in range(KB_S2):
                pltpu.make_async_copy(rows_v.at[pl.ds(j * 128, 128)],
                                      acc_sh.at[dv.at[j]], ssem).wait()
        return 0

    lax.fori_loop(0, n_blocks // 2, pair, 0)
    pltpu.make_async_copy(
        src2_hbm.at[cid, pl.ds(_al8(_r0(0) * 128), KB_S2 * 128)],
        srcv, isem0).wait()
    pltpu.make_async_copy(dst_hbm.at[pl.ds(_r0(0), KB_S2)], dstv,
                          isem0).wait()
    plsc.subcore_barrier()
    for q in range(28):
        qlo = _al8(lo + q * 112)
        pltpu.sync_copy(acc_sh.at[pl.ds(qlo, 112)], zbuf)
        pltpu.sync_copy(zbuf, out_hbm.at[cid, pl.ds(qlo, 112)])


_s2_call = functools.partial(
    pl.kernel,
    _s2_body,
    out_type=jax.ShapeDtypeStruct((NC, N_PAD, 32), jnp.float32),
    mesh=_mesh,
    scratch_types=[
        pltpu.VMEM_SHARED((N_PAD, 32), jnp.float32),
        pltpu.VMEM((112, 32), jnp.float32),
        pltpu.VMEM((KB_S2 * 128,), jnp.int32),
        pltpu.VMEM((KB_S2, 128), jnp.int32),
        pltpu.VMEM((KB_S2 * 128,), jnp.int32),
        pltpu.VMEM((KB_S2, 128), jnp.int32),
        pltpu.VMEM((KB_S2 * 128, 32), jnp.float32),
        pltpu.SemaphoreType.DMA,
        pltpu.SemaphoreType.DMA,
        pltpu.SemaphoreType.DMA,
        pltpu.SemaphoreType.DMA,
    ],
    compiler_params=pltpu.CompilerParams(use_tc_tiling_on_sc=False),
)()


def _prep_body(deg_ref, x12_ref, xd_ref, dinv_ref):
    deg = deg_ref[0, :] + deg_ref[1, :] + 1.0
    dinv = lax.rsqrt(deg)
    dinv_ref[...] = dinv
    xd12 = x12_ref[...] * dinv[:, None]
    xd_ref[...] = jnp.concatenate(
        [xd12, jnp.zeros((BN, 4), jnp.float32)], axis=1)


def _prep_call(deg2, x12):
    return pl.pallas_call(
        _prep_body,
        grid=(NBLK,),
        in_specs=[
            pl.BlockSpec((NC, BN), lambda i: (0, i)),
            pl.BlockSpec((BN, 12), lambda i: (i, 0)),
        ],
        out_specs=[
            pl.BlockSpec((BN, 16), lambda i: (i, 0)),
            pl.BlockSpec((BN,), lambda i: (i,)),
        ],
        out_shape=[
            jax.ShapeDtypeStruct((N_PAD, 16), jnp.float32),
            jax.ShapeDtypeStruct((N_PAD,), jnp.float32),
        ],
    )(deg2, x12)


def _mid_body(s1_ref, xd_ref, dinv_ref, w1_ref, b1_ref, k_ref, tb1_ref,
              x2d_ref):
    dinv = dinv_ref[...][:, None]
    y = (s1_ref[0] + s1_ref[1] + xd_ref[...]) * dinv
    w1 = w1_ref[...]
    b1 = b1_ref[...]

    def gcn(t):
        return jnp.maximum(
            jnp.dot(y[:, 4 * t:4 * t + 4], w1,
                    preferred_element_type=jnp.float32) + b1, 0.0)

    g9, g10, g11 = gcn(0), gcn(1), gcn(2)
    k0, k1, k2 = k_ref[0], k_ref[1], k_ref[2]
    tb1 = tb1_ref[...]
    o10 = jnp.maximum(
        jnp.dot(g9, k0, preferred_element_type=jnp.float32)
        + jnp.dot(g10, k1, preferred_element_type=jnp.float32)
        + jnp.dot(g11, k2, preferred_element_type=jnp.float32) + tb1, 0.0)
    o11 = jnp.maximum(
        jnp.dot(g10, k0, preferred_element_type=jnp.float32)
        + jnp.dot(g11, k1, preferred_element_type=jnp.float32) + tb1, 0.0)
    x2d_ref[0] = o10 * dinv
    x2d_ref[1] = o11 * dinv


def _mid_call(s1, xd16, dinv, w1, b1, tw1t, tb1):
    return pl.pallas_call(
        _mid_body,
        grid=(NBLK,),
        in_specs=[
            pl.BlockSpec((NC, BN, 16), lambda i: (0, i, 0)),
            pl.BlockSpec((BN, 16), lambda i: (i, 0)),
            pl.BlockSpec((BN,), lambda i: (i,)),
            pl.BlockSpec((F_IN, 32), lambda i: (0, 0)),
            pl.BlockSpec((32,), lambda i: (0,)),
            pl.BlockSpec((3, 32, 32), lambda i: (0, 0, 0)),
            pl.BlockSpec((32,), lambda i: (0,)),
        ],
        out_specs=pl.BlockSpec((2, BN, 32), lambda i: (0, i, 0)),
        out_shape=jax.ShapeDtypeStruct((2, N_PAD, 32), jnp.float32),
    )(s1, xd16, dinv, w1, b1, tw1t, tb1)


def _head_body(s2_ref, x2d_ref, dinv_ref, w2_ref, b2_ref, q_ref, tb2_ref,
               fw1_ref, fb1_ref, fw2_ref, fb2_ref, out_ref):
    dinv = dinv_ref[...][:, None]
    y10 = (s2_ref[0] + x2d_ref[0]) * dinv
    y11 = (s2_ref[1] + x2d_ref[1]) * dinv
    w2 = w2_ref[...]
    b2 = b2_ref[...]
    g10 = jnp.maximum(
        jnp.dot(y10, w2, preferred_element_type=jnp.float32) + b2, 0.0)
    g11 = jnp.maximum(
        jnp.dot(y11, w2, preferred_element_type=jnp.float32) + b2, 0.0)
    h = jnp.maximum(
        jnp.dot(g10, q_ref[0], preferred_element_type=jnp.float32)
        + jnp.dot(g11, q_ref[1], preferred_element_type=jnp.float32)
        + tb2_ref[...], 0.0)
    f = jnp.maximum(
        jnp.dot(h, fw1_ref[...], preferred_element_type=jnp.float32)
        + fb1_ref[...], 0.0)
    p = jnp.dot(f, fw2_ref[...], preferred_element_type=jnp.float32) \
        + fb2_ref[...]
    out_ref[...] = jnp.maximum(p, 0.0) + jnp.log1p(jnp.exp(-jnp.abs(p)))


def _head_call(s2, x2d, dinv, w2, b2, tw2t, tb2, fw1, fb1, fw2, fb2):
    return pl.pallas_call(
        _head_body,
        grid=(NBLK,),
        in_specs=[
            pl.BlockSpec((NC, BN, 32), lambda i: (0, i, 0)),
            pl.BlockSpec((2, BN, 32), lambda i: (0, i, 0)),
            pl.BlockSpec((BN,), lambda i: (i,)),
            pl.BlockSpec((32, 64), lambda i: (0, 0)),
            pl.BlockSpec((64,), lambda i: (0,)),
            pl.BlockSpec((2, 64, 64), lambda i: (0, 0, 0)),
            pl.BlockSpec((64,), lambda i: (0,)),
            pl.BlockSpec((64, 64), lambda i: (0, 0)),
            pl.BlockSpec((64,), lambda i: (0,)),
            pl.BlockSpec((64, HOR * F_IN), lambda i: (0, 0)),
            pl.BlockSpec((HOR * F_IN,), lambda i: (0,)),
        ],
        out_specs=pl.BlockSpec((BN, HOR * F_IN), lambda i: (i, 0)),
        out_shape=jax.ShapeDtypeStruct((N_PAD, HOR * F_IN), jnp.float32),
    )(s2, x2d, dinv, w2, b2, tw2t, tb2, fw1, fb1, fw2, fb2)


def kernel(x, edge_index, W1, b1, tw1, tb1, W2, b2, tw2, tb2,
           fw1, fb1, fw2, fb2):
    n = x.shape[0]
    x12 = x[:, T_STEPS - 3:, :].reshape(n, 3 * F_IN)
    x12 = jnp.pad(x12, ((0, N_PAD - n), (0, 0)))

    pad_n = E_PAD - E_EDGES
    ar = jnp.arange(pad_n, dtype=jnp.int32)
    src_flat = jnp.concatenate([edge_index[0], ar % N_NODES])
    dstp = jnp.concatenate(
        [edge_index[1], N_NODES + (ar % (N_PAD - N_NODES))]).reshape(
        IDX_ROWS, 128)
    src2 = jnp.stack([src_flat, src_flat + N_PAD])

    tw1t = jnp.transpose(tw1, (2, 1, 0))
    tw2t = jnp.transpose(tw2, (2, 1, 0))[:2]

    deg2 = _deg_call(dstp).reshape(NC, N_PAD)
    xd16, dinv = _prep_call(deg2, x12)
    s1 = _s1_call(xd16, src_flat, dstp)
    x2d = _mid_call(s1, xd16, dinv, W1, b1, tw1t, tb1)
    tab2 = x2d.reshape(2 * N_PAD, 32)
    s2 = _s2_call(tab2, src2, dstp)
    out = _head_call(s2, x2d, dinv, W2, b2, tw2t, tb2, fw1, fb1, fw2, fb2)
    return out[:n].reshape(n, HOR, F_IN)
